# baseline (speedup 1.0000x reference)
import functools

import jax
import jax.numpy as jnp
from jax import lax
from jax.experimental import pallas as pl
from jax.experimental.pallas import tpu as pltpu
from jax.experimental.pallas import tpu_sc as plsc

K_TOP = 20
CHUNK = 128
NEG_INF = float("-inf")


def _tc_kernel(q_ref, db_ref, sim_ref, m2_ref, s2_ref, mmax_ref,
               buf_scr, *, nb, num_n):
    j = pl.program_id(1)
    tile = lax.dot_general(
        q_ref[...], db_ref[...],
        dimension_numbers=(((1,), (1,)), ((), ())),
        preferred_element_type=jnp.float32,
    )
    qb = tile.shape[0]
    nch = nb // CHUNK

    for c in range(nch):
        piece = tile[:, c * CHUNK:(c + 1) * CHUNK]
        sim_ref[:, c, :] = piece
    buf_scr[:, pl.ds(j * nb, nb)] = tile

    @pl.when(j == num_n - 1)
    def _():
        buf = buf_scr[...]
        n = buf.shape[1]
        cms = []
        for c in range(n // CHUNK):
            cms.append(jnp.max(buf[:, c * CHUNK:(c + 1) * CHUNK], axis=1,
                               keepdims=True))
        cm = jnp.concatenate(cms, axis=1)
        mmax_ref[...] = cm
        m = jnp.max(cm, axis=1, keepdims=True)
        s = jnp.sum(jnp.exp(buf - m), axis=1, keepdims=True)
        m2_ref[...] = jnp.broadcast_to(m, (qb, 128))
        s2_ref[...] = jnp.broadcast_to(s, (qb, 128))


def _tc_stage(queries, db):
    q_n, d = queries.shape
    n, _ = db.shape
    qb = 128
    nb = 2048
    num_q = q_n // qb
    num_n = n // nb
    n_chunks = n // CHUNK

    return pl.pallas_call(
        functools.partial(_tc_kernel, nb=nb, num_n=num_n),
        grid=(num_q, num_n),
        in_specs=[
            pl.BlockSpec((qb, d), lambda i, j: (i, 0)),
            pl.BlockSpec((nb, d), lambda i, j: (j, 0)),
        ],
        out_specs=[
            pl.BlockSpec((qb, nb // CHUNK, CHUNK), lambda i, j: (i, j, 0)),
            pl.BlockSpec((qb, 128), lambda i, j: (i, 0)),
            pl.BlockSpec((qb, 128), lambda i, j: (i, 0)),
            pl.BlockSpec((qb, n_chunks), lambda i, j: (i, 0)),
        ],
        out_shape=[
            jax.ShapeDtypeStruct((q_n, n_chunks, CHUNK), jnp.float32),
            jax.ShapeDtypeStruct((q_n, 128), jnp.float32),
            jax.ShapeDtypeStruct((q_n, 128), jnp.float32),
            jax.ShapeDtypeStruct((q_n, n_chunks), jnp.float32),
        ],
        scratch_shapes=[
            pltpu.VMEM((qb, n), jnp.float32),
        ],
        compiler_params=pltpu.CompilerParams(
            dimension_semantics=("arbitrary", "arbitrary"),
        ),
    )(queries, db)


def _cmpsel(av, ai, bv, bi):
    ge = av >= bv
    hv = jnp.where(ge, av, bv)
    hi = jnp.where(ge, ai, bi)
    lv = jnp.where(ge, bv, av)
    li = jnp.where(ge, bi, ai)
    return hv, hi, lv, li


def _rev(x):
    return lax.rev(x, (0,))


def _sort16(v, i):
    return plsc.sort_key_val(v, i, descending=True)


def _sort2_full(av, ai, bv, bi):
    rbv, rbi = _rev(bv), _rev(bi)
    hv, hi, lv, li = _cmpsel(av, ai, rbv, rbi)
    r0 = _sort16(hv, hi)
    r1 = _sort16(lv, li)
    return r0[0], r0[1], r1[0], r1[1]


def _merge32(av0, ai0, av1, ai1, bv0, bi0, bv1, bi1):
    h0v, h0i, _, _ = _cmpsel(av0, ai0, _rev(bv1), _rev(bi1))
    h1v, h1i, _, _ = _cmpsel(av1, ai1, _rev(bv0), _rev(bi0))
    ev, ei, fv, fi = _cmpsel(h0v, h0i, h1v, h1i)
    r0 = _sort16(ev, ei)
    r1 = _sort16(fv, fi)
    return r0[0], r0[1], r1[0], r1[1]


def _sc_stage(sim2d, mmax, m2, s2, *, q_n, n_chunks, k):
    info = plsc.get_sparse_core_info()
    nw = info.num_cores * info.num_subcores
    rows_per_w = q_n // nw
    n_groups = n_chunks // 16
    mesh = plsc.VectorSubcoreMesh(core_axis_name="c", subcore_axis_name="s")

    @functools.partial(
        pl.kernel,
        mesh=mesh,
        out_type=[
            jax.ShapeDtypeStruct((q_n, 32), jnp.float32),
            jax.ShapeDtypeStruct((q_n, 32), jnp.int32),
        ],
        scratch_types=[
            pltpu.VMEM((rows_per_w, n_chunks), jnp.float32),
            pltpu.VMEM((rows_per_w, 128), jnp.float32),
            pltpu.VMEM((rows_per_w, 128), jnp.float32),
            pltpu.VMEM((32, CHUNK), jnp.float32),
            pltpu.VMEM((rows_per_w, 32), jnp.float32),
            pltpu.VMEM((rows_per_w, 32), jnp.int32),
            pltpu.SemaphoreType.DMA,
        ],
        compiler_params=pltpu.CompilerParams(needs_layout_passes=False),
    )
    def body(sim_hbm, mm_hbm, m2_hbm, s2_hbm, osc_hbm, oix_hbm,
             mv, msv, ssv, gbuf, ov, oi, sem):
        wid = lax.axis_index("s") * info.num_cores + lax.axis_index("c")
        base = wid * rows_per_w
        pltpu.sync_copy(mm_hbm.at[pl.ds(base, rows_per_w)], mv)
        pltpu.sync_copy(m2_hbm.at[pl.ds(base, rows_per_w)], msv)
        pltpu.sync_copy(s2_hbm.at[pl.ds(base, rows_per_w)], ssv)

        iota = lax.iota(jnp.int32, 16)
        neg_inf_v = jnp.full((16,), NEG_INF, jnp.float32)

        def row_body(i, carry):
            r = base + i
            m_splat = msv[i, pl.ds(0, 16)]
            s_splat = ssv[i, pl.ds(0, 16)]

            lists = []
            for g in range(n_groups):
                v = mv[i, pl.ds(g * 16, 16)]
                ids = iota + (g * 16)
                lists.append(_sort16(v, ids))
            l32 = []
            for p in range(n_groups // 2):
                a, b = lists[2 * p], lists[2 * p + 1]
                l32.append(_sort2_full(a[0], a[1], b[0], b[1]))
            while len(l32) > 1:
                nxt = []
                for p in range(len(l32) // 2):
                    nxt.append(_merge32(*l32[2 * p], *l32[2 * p + 1]))
                l32 = nxt
            tv0, ti0, tv1, ti1 = l32[0]

            t_splat = jnp.take(tv1, jnp.full((16,), k - 17, jnp.int32))
            cnt = (jnp.max(plsc.all_reduce_population_count(tv0 >= t_splat))
                   + jnp.max(plsc.all_reduce_population_count(tv1 >= t_splat)))

            row_off = r * n_chunks
            c1 = pltpu.async_copy(sim_hbm.at[row_off + ti0],
                                  gbuf.at[pl.ds(0, 16)], sem)
            c2 = pltpu.async_copy(sim_hbm.at[row_off + ti1],
                                  gbuf.at[pl.ds(16, 16)], sem)
            c1.wait()
            c2.wait()

            rv0, ri0 = neg_inf_v, jnp.zeros((16,), jnp.int32)
            rv1, ri1 = neg_inf_v, jnp.zeros((16,), jnp.int32)
            for q in range(32):
                ti = ti0 if q < 16 else ti1
                cid = jnp.take(ti, jnp.full((16,), q % 16, jnp.int32))
                colbase = cid * CHUNK
                valid = q < cnt
                for jj in range(CHUNK // 16):
                    v = gbuf[q, pl.ds(jj * 16, 16)]
                    gidx = colbase + (jj * 16) + iota
                    mask = (v >= t_splat) & jnp.full((16,), valid, jnp.bool_)
                    pc = jnp.max(plsc.all_reduce_population_count(mask))

                    def do_merge(rv0=rv0, ri0=ri0, rv1=rv1, ri1=ri1,
                                 v=v, gidx=gidx, mask=mask):
                        sv, si = _sort16(jnp.where(mask, v, NEG_INF), gidx)
                        return _merge32(rv0, ri0, rv1, ri1,
                                        sv, si, neg_inf_v,
                                        jnp.zeros((16,), jnp.int32))

                    def no_merge(rv0=rv0, ri0=ri0, rv1=rv1, ri1=ri1):
                        return rv0, ri0, rv1, ri1

                    rv0, ri0, rv1, ri1 = lax.cond(pc > 0, do_merge, no_merge)

            big = jnp.int32(2 ** 30)
            sc_a = neg_inf_v
            sc_b = neg_inf_v
            ix_a = jnp.zeros((16,), jnp.int32)
            ix_b = jnp.zeros((16,), jnp.int32)
            for rnd in range(k):
                gm = jnp.max(jnp.maximum(rv0, rv1))
                gm_splat = jnp.full((16,), gm, jnp.float32)
                c0 = rv0 == gm_splat
                c1m = rv1 == gm_splat
                gidx = jnp.min(jnp.minimum(jnp.where(c0, ri0, big),
                                           jnp.where(c1m, ri1, big)))
                gidx_splat = jnp.full((16,), gidx, jnp.int32)
                score = jnp.exp(gm_splat - m_splat) / s_splat
                lane = iota == (rnd % 16)
                if rnd < 16:
                    sc_a = jnp.where(lane, score, sc_a)
                    ix_a = jnp.where(lane, gidx_splat, ix_a)
                else:
                    sc_b = jnp.where(lane, score, sc_b)
                    ix_b = jnp.where(lane, gidx_splat, ix_b)
                rv0 = jnp.where(c0 & (ri0 == gidx_splat), NEG_INF, rv0)
                rv1 = jnp.where(c1m & (ri1 == gidx_splat), NEG_INF, rv1)

            ov[i, pl.ds(0, 16)] = sc_a
            ov[i, pl.ds(16, 16)] = sc_b
            oi[i, pl.ds(0, 16)] = ix_a
            oi[i, pl.ds(16, 16)] = ix_b
            return carry

        lax.fori_loop(0, rows_per_w, row_body, jnp.int32(0))

        pltpu.sync_copy(ov, osc_hbm.at[pl.ds(base, rows_per_w)])
        pltpu.sync_copy(oi, oix_hbm.at[pl.ds(base, rows_per_w)])

    return body(sim2d, mmax, m2, s2)


@jax.jit
def kernel(queries, db):
    q_n, _ = queries.shape
    n, _ = db.shape
    k = min(K_TOP, n)
    n_chunks = n // CHUNK

    sim3d, m2, s2, mmax = _tc_stage(queries, db)
    sim2d = jnp.reshape(sim3d, (q_n * n_chunks, CHUNK))
    oscores, oinds = _sc_stage(sim2d, mmax, m2, s2,
                               q_n=q_n, n_chunks=n_chunks, k=k)

    rows = jnp.arange(q_n * k, dtype=jnp.int32) // k
    return rows, oinds[:, :k].reshape(-1), oscores[:, :k].reshape(-1)


# speedup vs baseline: 1.3650x; 1.3650x over previous
import functools

import jax
import jax.numpy as jnp
from jax import lax
from jax.experimental import pallas as pl
from jax.experimental.pallas import tpu as pltpu
from jax.experimental.pallas import tpu_sc as plsc

K_TOP = 20
CHUNK = 128
NEG_INF = float("-inf")


def _tc_kernel(q_ref, db_ref, sim_ref, m2_ref, s2_ref, mmax_ref,
               buf_scr, *, nb, num_n):
    j = pl.program_id(1)
    tile = lax.dot_general(
        q_ref[...], db_ref[...],
        dimension_numbers=(((1,), (1,)), ((), ())),
        preferred_element_type=jnp.float32,
    )
    qb = tile.shape[0]
    nch = nb // CHUNK

    for c in range(nch):
        piece = tile[:, c * CHUNK:(c + 1) * CHUNK]
        sim_ref[:, c, :] = piece
    buf_scr[:, pl.ds(j * nb, nb)] = tile

    @pl.when(j == num_n - 1)
    def _():
        buf = buf_scr[...]
        n = buf.shape[1]
        cms = []
        for c in range(n // CHUNK):
            cms.append(jnp.max(buf[:, c * CHUNK:(c + 1) * CHUNK], axis=1,
                               keepdims=True))
        cm = jnp.concatenate(cms, axis=1)
        mmax_ref[...] = cm
        m = jnp.max(cm, axis=1, keepdims=True)
        s = jnp.sum(jnp.exp(buf - m), axis=1, keepdims=True)
        m2_ref[...] = jnp.broadcast_to(m, (qb, 128))
        s2_ref[...] = jnp.broadcast_to(s, (qb, 128))


def _tc_stage(queries, db):
    q_n, d = queries.shape
    n, _ = db.shape
    qb = 128
    nb = 2048
    num_q = q_n // qb
    num_n = n // nb
    n_chunks = n // CHUNK

    return pl.pallas_call(
        functools.partial(_tc_kernel, nb=nb, num_n=num_n),
        grid=(num_q, num_n),
        in_specs=[
            pl.BlockSpec((qb, d), lambda i, j: (i, 0)),
            pl.BlockSpec((nb, d), lambda i, j: (j, 0)),
        ],
        out_specs=[
            pl.BlockSpec((qb, nb // CHUNK, CHUNK), lambda i, j: (i, j, 0)),
            pl.BlockSpec((qb, 128), lambda i, j: (i, 0)),
            pl.BlockSpec((qb, 128), lambda i, j: (i, 0)),
            pl.BlockSpec((qb, n_chunks), lambda i, j: (i, 0)),
        ],
        out_shape=[
            jax.ShapeDtypeStruct((q_n, n_chunks, CHUNK), jnp.float32),
            jax.ShapeDtypeStruct((q_n, 128), jnp.float32),
            jax.ShapeDtypeStruct((q_n, 128), jnp.float32),
            jax.ShapeDtypeStruct((q_n, n_chunks), jnp.float32),
        ],
        scratch_shapes=[
            pltpu.VMEM((qb, n), jnp.float32),
        ],
        compiler_params=pltpu.CompilerParams(
            dimension_semantics=("arbitrary", "arbitrary"),
        ),
    )(queries, db)


def _cmpsel(av, ai, bv, bi):
    ge = av >= bv
    hv = jnp.where(ge, av, bv)
    hi = jnp.where(ge, ai, bi)
    lv = jnp.where(ge, bv, av)
    li = jnp.where(ge, bi, ai)
    return hv, hi, lv, li


def _rev(x):
    return lax.rev(x, (0,))


def _sort16(v, i):
    return plsc.sort_key_val(v, i, descending=True)


def _sort2_full(av, ai, bv, bi):
    rbv, rbi = _rev(bv), _rev(bi)
    hv, hi, lv, li = _cmpsel(av, ai, rbv, rbi)
    r0 = _sort16(hv, hi)
    r1 = _sort16(lv, li)
    return r0[0], r0[1], r1[0], r1[1]


def _merge32(av0, ai0, av1, ai1, bv0, bi0, bv1, bi1):
    h0v, h0i, _, _ = _cmpsel(av0, ai0, _rev(bv1), _rev(bi1))
    h1v, h1i, _, _ = _cmpsel(av1, ai1, _rev(bv0), _rev(bi0))
    ev, ei, fv, fi = _cmpsel(h0v, h0i, h1v, h1i)
    r0 = _sort16(ev, ei)
    r1 = _sort16(fv, fi)
    return r0[0], r0[1], r1[0], r1[1]


def _sc_stage(sim2d, mmax, m2, s2, *, q_n, n_chunks, k):
    info = plsc.get_sparse_core_info()
    nw = info.num_cores * info.num_subcores
    rows_per_w = q_n // nw
    n_groups = n_chunks // 16
    mesh = plsc.VectorSubcoreMesh(core_axis_name="c", subcore_axis_name="s")

    @functools.partial(
        pl.kernel,
        mesh=mesh,
        out_type=[
            jax.ShapeDtypeStruct((q_n, 32), jnp.float32),
            jax.ShapeDtypeStruct((q_n, 32), jnp.int32),
        ],
        scratch_types=[
            pltpu.VMEM((rows_per_w, n_chunks), jnp.float32),
            pltpu.VMEM((rows_per_w, 128), jnp.float32),
            pltpu.VMEM((rows_per_w, 128), jnp.float32),
            pltpu.VMEM((32, CHUNK), jnp.float32),
            pltpu.VMEM((32, CHUNK), jnp.float32),
            pltpu.VMEM((rows_per_w, 32), jnp.float32),
            pltpu.VMEM((rows_per_w, 32), jnp.int32),
            pltpu.SemaphoreType.DMA,
            pltpu.SemaphoreType.DMA,
        ],
        compiler_params=pltpu.CompilerParams(needs_layout_passes=False),
    )
    def body(sim_hbm, mm_hbm, m2_hbm, s2_hbm, osc_hbm, oix_hbm,
             mv, msv, ssv, gbuf_a, gbuf_b, ov, oi, sem_a, sem_b):
        wid = lax.axis_index("s") * info.num_cores + lax.axis_index("c")
        base = wid * rows_per_w
        pltpu.sync_copy(mm_hbm.at[pl.ds(base, rows_per_w)], mv)
        pltpu.sync_copy(m2_hbm.at[pl.ds(base, rows_per_w)], msv)
        pltpu.sync_copy(s2_hbm.at[pl.ds(base, rows_per_w)], ssv)

        iota = lax.iota(jnp.int32, 16)
        neg_inf_v = jnp.full((16,), NEG_INF, jnp.float32)
        zero_i = jnp.zeros((16,), jnp.int32)

        def phase_a(i, gbuf, sem):
            lists = []
            for g in range(n_groups):
                v = mv[i, pl.ds(g * 16, 16)]
                ids = iota + (g * 16)
                lists.append(_sort16(v, ids))
            l32 = []
            for p in range(n_groups // 2):
                a, b = lists[2 * p], lists[2 * p + 1]
                l32.append(_sort2_full(a[0], a[1], b[0], b[1]))
            while len(l32) > 1:
                nxt = []
                for p in range(len(l32) // 2):
                    nxt.append(_merge32(*l32[2 * p], *l32[2 * p + 1]))
                l32 = nxt
            tv0, ti0, tv1, ti1 = l32[0]

            t_splat = jnp.take(tv1, jnp.full((16,), k - 17, jnp.int32))
            cnt = (jnp.max(plsc.all_reduce_population_count(tv0 >= t_splat))
                   + jnp.max(plsc.all_reduce_population_count(tv1 >= t_splat)))

            row_off = (base + i) * n_chunks
            pltpu.async_copy(sim_hbm.at[row_off + ti0],
                             gbuf.at[pl.ds(0, 16)], sem)
            pltpu.async_copy(sim_hbm.at[row_off + ti1],
                             gbuf.at[pl.ds(16, 16)], sem)
            return ti0, ti1, t_splat, cnt

        def process(i, gbuf, sem, st):
            ti0, ti1, t_splat, cnt = st
            m_splat = msv[i, pl.ds(0, 16)]
            s_splat = ssv[i, pl.ds(0, 16)]

            pltpu.make_async_copy(sim_hbm.at[pl.ds(0, 32)], gbuf, sem).wait()

            def chunk_body(q, carry):
                rv0, ri0, rv1, ri1 = carry
                use0 = jnp.full((16,), q < 16, jnp.bool_)
                ti = jnp.where(use0, ti0, ti1)
                cid = jnp.take(ti, jnp.full((16,), q % 16, jnp.int32))
                colbase = cid * CHUNK
                validv = jnp.full((16,), q < cnt, jnp.bool_)
                for jj in range(CHUNK // 16):
                    v = gbuf[q, pl.ds(jj * 16, 16)]
                    gidx = colbase + (jj * 16) + iota
                    mask = (v >= t_splat) & validv
                    pc = jnp.max(plsc.all_reduce_population_count(mask))

                    def do_merge(rv0=rv0, ri0=ri0, rv1=rv1, ri1=ri1,
                                 v=v, gidx=gidx, mask=mask):
                        sv, si = _sort16(jnp.where(mask, v, NEG_INF), gidx)
                        return _merge32(rv0, ri0, rv1, ri1,
                                        sv, si, neg_inf_v, zero_i)

                    def no_merge(rv0=rv0, ri0=ri0, rv1=rv1, ri1=ri1):
                        return rv0, ri0, rv1, ri1

                    rv0, ri0, rv1, ri1 = lax.cond(pc > 0, do_merge, no_merge)
                return rv0, ri0, rv1, ri1

            rv0, ri0, rv1, ri1 = lax.fori_loop(
                0, 32, chunk_body, (neg_inf_v, zero_i, neg_inf_v, zero_i))

            sh0 = jnp.where(iota == 15,
                            jnp.take(rv1, jnp.full((16,), 0, jnp.int32)),
                            jnp.take(rv0, jnp.minimum(iota + 1, 15)))
            sh1 = jnp.take(rv1, jnp.minimum(iota + 1, 15))
            t0 = jnp.max(plsc.all_reduce_population_count(rv0 == sh0))
            t1 = jnp.max(plsc.all_reduce_population_count(
                (rv1 == sh1) & (iota < 5)))

            def fast(rv0=rv0, ri0=ri0, rv1=rv1, ri1=ri1):
                sc0 = jnp.exp(rv0 - m_splat) / s_splat
                sc1 = jnp.exp(rv1 - m_splat) / s_splat
                return sc0, ri0, sc1, ri1

            def slow(rv0=rv0, ri0=ri0, rv1=rv1, ri1=ri1):
                big = jnp.int32(2 ** 30)
                sc_a = neg_inf_v
                sc_b = neg_inf_v
                ix_a = zero_i
                ix_b = zero_i
                for rnd in range(k):
                    gm = jnp.max(jnp.maximum(rv0, rv1))
                    gm_splat = jnp.full((16,), gm, jnp.float32)
                    c0 = rv0 == gm_splat
                    c1m = rv1 == gm_splat
                    gidx = jnp.min(jnp.minimum(jnp.where(c0, ri0, big),
                                               jnp.where(c1m, ri1, big)))
                    gidx_splat = jnp.full((16,), gidx, jnp.int32)
                    score = jnp.exp(gm_splat - m_splat) / s_splat
                    lane = iota == (rnd % 16)
                    if rnd < 16:
                        sc_a = jnp.where(lane, score, sc_a)
                        ix_a = jnp.where(lane, gidx_splat, ix_a)
                    else:
                        sc_b = jnp.where(lane, score, sc_b)
                        ix_b = jnp.where(lane, gidx_splat, ix_b)
                    rv0 = jnp.where(c0 & (ri0 == gidx_splat), NEG_INF, rv0)
                    rv1 = jnp.where(c1m & (ri1 == gidx_splat), NEG_INF, rv1)
                return sc_a, ix_a, sc_b, ix_b

            sc_a, ix_a, sc_b, ix_b = lax.cond(t0 + t1 > 0, slow, fast)

            ov[i, pl.ds(0, 16)] = sc_a
            ov[i, pl.ds(16, 16)] = sc_b
            oi[i, pl.ds(0, 16)] = ix_a
            oi[i, pl.ds(16, 16)] = ix_b

        st_a0 = phase_a(jnp.int32(0), gbuf_a, sem_a)
        st_b0 = phase_a(jnp.int32(1), gbuf_b, sem_b)

        def pipe_body(t, carry):
            st_a, st_b = carry
            i = 2 * t
            process(i, gbuf_a, sem_a, st_a)
            st_a = phase_a(i + 2, gbuf_a, sem_a)
            process(i + 1, gbuf_b, sem_b, st_b)
            st_b = phase_a(i + 3, gbuf_b, sem_b)
            return st_a, st_b

        st_a, st_b = lax.fori_loop(0, rows_per_w // 2 - 1, pipe_body,
                                   (st_a0, st_b0))
        process(jnp.int32(rows_per_w - 2), gbuf_a, sem_a, st_a)
        process(jnp.int32(rows_per_w - 1), gbuf_b, sem_b, st_b)

        pltpu.sync_copy(ov, osc_hbm.at[pl.ds(base, rows_per_w)])
        pltpu.sync_copy(oi, oix_hbm.at[pl.ds(base, rows_per_w)])

    return body(sim2d, mmax, m2, s2)


@jax.jit
def kernel(queries, db):
    q_n, _ = queries.shape
    n, _ = db.shape
    k = min(K_TOP, n)
    n_chunks = n // CHUNK

    sim3d, m2, s2, mmax = _tc_stage(queries, db)
    sim2d = jnp.reshape(sim3d, (q_n * n_chunks, CHUNK))
    oscores, oinds = _sc_stage(sim2d, mmax, m2, s2,
                               q_n=q_n, n_chunks=n_chunks, k=k)

    rows = jnp.arange(q_n * k, dtype=jnp.int32) // k
    return rows, oinds[:, :k].reshape(-1), oscores[:, :k].reshape(-1)


# speedup vs baseline: 1.5138x; 1.1090x over previous
import functools

import jax
import jax.numpy as jnp
from jax import lax
from jax.experimental import pallas as pl
from jax.experimental.pallas import tpu as pltpu
from jax.experimental.pallas import tpu_sc as plsc

K_TOP = 20
CHUNK = 128
NEG_INF = float("-inf")


def _tc_kernel(q_ref, db_ref, sim_ref, m2_ref, s2_ref, mmax_ref,
               buf_scr, *, nb, num_n):
    j = pl.program_id(1)
    tile = lax.dot_general(
        q_ref[...], db_ref[...],
        dimension_numbers=(((1,), (1,)), ((), ())),
        preferred_element_type=jnp.float32,
    )
    qb = tile.shape[0]
    nch = nb // CHUNK

    for c in range(nch):
        piece = tile[:, c * CHUNK:(c + 1) * CHUNK]
        sim_ref[:, c, :] = piece
    buf_scr[:, pl.ds(j * nb, nb)] = tile

    @pl.when(j == num_n - 1)
    def _():
        buf = buf_scr[...]
        n = buf.shape[1]
        cms = []
        for c in range(n // CHUNK):
            cms.append(jnp.max(buf[:, c * CHUNK:(c + 1) * CHUNK], axis=1,
                               keepdims=True))
        cm = jnp.concatenate(cms, axis=1)
        mmax_ref[...] = cm
        m = jnp.max(cm, axis=1, keepdims=True)
        s = jnp.sum(jnp.exp(buf - m), axis=1, keepdims=True)
        m2_ref[...] = jnp.broadcast_to(m, (qb, 128))
        s2_ref[...] = jnp.broadcast_to(s, (qb, 128))


def _tc_stage(queries, db):
    q_n, d = queries.shape
    n, _ = db.shape
    qb = 128
    nb = 2048
    num_q = q_n // qb
    num_n = n // nb
    n_chunks = n // CHUNK

    return pl.pallas_call(
        functools.partial(_tc_kernel, nb=nb, num_n=num_n),
        grid=(num_q, num_n),
        in_specs=[
            pl.BlockSpec((qb, d), lambda i, j: (i, 0)),
            pl.BlockSpec((nb, d), lambda i, j: (j, 0)),
        ],
        out_specs=[
            pl.BlockSpec((qb, nb // CHUNK, CHUNK), lambda i, j: (i, j, 0)),
            pl.BlockSpec((qb, 128), lambda i, j: (i, 0)),
            pl.BlockSpec((qb, 128), lambda i, j: (i, 0)),
            pl.BlockSpec((qb, n_chunks), lambda i, j: (i, 0)),
        ],
        out_shape=[
            jax.ShapeDtypeStruct((q_n, n_chunks, CHUNK), jnp.float32),
            jax.ShapeDtypeStruct((q_n, 128), jnp.float32),
            jax.ShapeDtypeStruct((q_n, 128), jnp.float32),
            jax.ShapeDtypeStruct((q_n, n_chunks), jnp.float32),
        ],
        scratch_shapes=[
            pltpu.VMEM((qb, n), jnp.float32),
        ],
        compiler_params=pltpu.CompilerParams(
            dimension_semantics=("arbitrary", "arbitrary"),
        ),
    )(queries, db)


def _cmpsel(av, ai, bv, bi):
    ge = av >= bv
    hv = jnp.where(ge, av, bv)
    hi = jnp.where(ge, ai, bi)
    lv = jnp.where(ge, bv, av)
    li = jnp.where(ge, bi, ai)
    return hv, hi, lv, li


def _rev(x):
    return lax.rev(x, (0,))


def _sort16(v, i):
    return plsc.sort_key_val(v, i, descending=True)


def _sort2_full(av, ai, bv, bi):
    rbv, rbi = _rev(bv), _rev(bi)
    hv, hi, lv, li = _cmpsel(av, ai, rbv, rbi)
    r0 = _sort16(hv, hi)
    r1 = _sort16(lv, li)
    return r0[0], r0[1], r1[0], r1[1]


def _merge32(av0, ai0, av1, ai1, bv0, bi0, bv1, bi1):
    h0v, h0i, _, _ = _cmpsel(av0, ai0, _rev(bv1), _rev(bi1))
    h1v, h1i, _, _ = _cmpsel(av1, ai1, _rev(bv0), _rev(bi0))
    ev, ei, fv, fi = _cmpsel(h0v, h0i, h1v, h1i)
    r0 = _sort16(ev, ei)
    r1 = _sort16(fv, fi)
    return r0[0], r0[1], r1[0], r1[1]


def _sc_stage(sim2d, mmax, m2, s2, *, q_n, n_chunks, k):
    info = plsc.get_sparse_core_info()
    nw = info.num_cores * info.num_subcores
    rows_per_w = q_n // nw
    n_groups = n_chunks // 16
    mesh = plsc.VectorSubcoreMesh(core_axis_name="c", subcore_axis_name="s")

    @functools.partial(
        pl.kernel,
        mesh=mesh,
        out_type=[
            jax.ShapeDtypeStruct((q_n, 32), jnp.float32),
            jax.ShapeDtypeStruct((q_n, 32), jnp.int32),
        ],
        scratch_types=[
            pltpu.VMEM((rows_per_w, n_chunks), jnp.float32),
            pltpu.VMEM((rows_per_w, 128), jnp.float32),
            pltpu.VMEM((rows_per_w, 128), jnp.float32),
            pltpu.VMEM((32, CHUNK), jnp.float32),
            pltpu.VMEM((32, CHUNK), jnp.float32),
            pltpu.VMEM((rows_per_w, 32), jnp.float32),
            pltpu.VMEM((rows_per_w, 32), jnp.int32),
            pltpu.SemaphoreType.DMA,
            pltpu.SemaphoreType.DMA,
        ],
        compiler_params=pltpu.CompilerParams(needs_layout_passes=False),
    )
    def body(sim_hbm, mm_hbm, m2_hbm, s2_hbm, osc_hbm, oix_hbm,
             mv, msv, ssv, gbuf_a, gbuf_b, ov, oi, sem_a, sem_b):
        wid = lax.axis_index("s") * info.num_cores + lax.axis_index("c")
        base = wid * rows_per_w
        pltpu.sync_copy(mm_hbm.at[pl.ds(base, rows_per_w)], mv)
        pltpu.sync_copy(m2_hbm.at[pl.ds(base, rows_per_w)], msv)
        pltpu.sync_copy(s2_hbm.at[pl.ds(base, rows_per_w)], ssv)

        iota = lax.iota(jnp.int32, 16)
        neg_inf_v = jnp.full((16,), NEG_INF, jnp.float32)
        zero_i = jnp.zeros((16,), jnp.int32)

        def phase_a(i, gbuf, sem):
            lists = []
            for g in range(n_groups):
                v = mv[i, pl.ds(g * 16, 16)]
                ids = iota + (g * 16)
                lists.append(_sort16(v, ids))
            l32 = []
            for p in range(n_groups // 2):
                a, b = lists[2 * p], lists[2 * p + 1]
                l32.append(_sort2_full(a[0], a[1], b[0], b[1]))
            while len(l32) > 1:
                nxt = []
                for p in range(len(l32) // 2):
                    nxt.append(_merge32(*l32[2 * p], *l32[2 * p + 1]))
                l32 = nxt
            tv0, ti0, tv1, ti1 = l32[0]

            t_splat = jnp.take(tv1, jnp.full((16,), k - 17, jnp.int32))
            cnt = (16 + plsc.all_reduce_population_count(tv1 >= t_splat)[0])

            row_off = (base + i) * n_chunks
            pltpu.async_copy(sim_hbm.at[row_off + ti0],
                             gbuf.at[pl.ds(0, 16)], sem)
            pltpu.async_copy(sim_hbm.at[row_off + ti1],
                             gbuf.at[pl.ds(16, 16)], sem)
            return ti0, ti1, t_splat, cnt

        def process(i, gbuf, sem, st):
            ti0, ti1, t_splat, cnt = st
            m_splat = msv[i, pl.ds(0, 16)]
            s_splat = ssv[i, pl.ds(0, 16)]

            pltpu.make_async_copy(sim_hbm.at[pl.ds(0, 32)], gbuf, sem).wait()

            def chunk_body(q, carry):
                rv0, ri0, rv1, ri1 = carry
                use0 = jnp.full((16,), q < 16, jnp.bool_)
                ti = jnp.where(use0, ti0, ti1)
                cid = jnp.take(ti, jnp.full((16,), q % 16, jnp.int32))
                colbase = cid * CHUNK
                for jj in range(CHUNK // 16):
                    v = gbuf[q, pl.ds(jj * 16, 16)]
                    gidx = colbase + (jj * 16) + iota
                    mask = v >= t_splat
                    pc = plsc.all_reduce_population_count(mask)[0]

                    def do_merge(rv0=rv0, ri0=ri0, rv1=rv1, ri1=ri1,
                                 v=v, gidx=gidx, mask=mask):
                        sv, si = _sort16(jnp.where(mask, v, NEG_INF), gidx)
                        return _merge32(rv0, ri0, rv1, ri1,
                                        sv, si, neg_inf_v, zero_i)

                    def no_merge(rv0=rv0, ri0=ri0, rv1=rv1, ri1=ri1):
                        return rv0, ri0, rv1, ri1

                    rv0, ri0, rv1, ri1 = lax.cond(pc > 0, do_merge, no_merge)
                return rv0, ri0, rv1, ri1

            rv0, ri0, rv1, ri1 = lax.fori_loop(
                0, cnt, chunk_body, (neg_inf_v, zero_i, neg_inf_v, zero_i))

            sh0 = jnp.where(iota == 15,
                            jnp.take(rv1, jnp.full((16,), 0, jnp.int32)),
                            jnp.take(rv0, jnp.minimum(iota + 1, 15)))
            sh1 = jnp.take(rv1, jnp.minimum(iota + 1, 15))
            t0 = plsc.all_reduce_population_count(rv0 == sh0)[0]
            t1 = plsc.all_reduce_population_count(
                (rv1 == sh1) & (iota < 5))[0]

            def fast(rv0=rv0, ri0=ri0, rv1=rv1, ri1=ri1):
                sc0 = jnp.exp(rv0 - m_splat) / s_splat
                sc1 = jnp.exp(rv1 - m_splat) / s_splat
                return sc0, ri0, sc1, ri1

            def slow(rv0=rv0, ri0=ri0, rv1=rv1, ri1=ri1):
                big = jnp.int32(2 ** 30)
                sc_a = neg_inf_v
                sc_b = neg_inf_v
                ix_a = zero_i
                ix_b = zero_i
                for rnd in range(k):
                    gm = jnp.max(jnp.maximum(rv0, rv1))
                    gm_splat = jnp.full((16,), gm, jnp.float32)
                    c0 = rv0 == gm_splat
                    c1m = rv1 == gm_splat
                    gidx = jnp.min(jnp.minimum(jnp.where(c0, ri0, big),
                                               jnp.where(c1m, ri1, big)))
                    gidx_splat = jnp.full((16,), gidx, jnp.int32)
                    score = jnp.exp(gm_splat - m_splat) / s_splat
                    lane = iota == (rnd % 16)
                    if rnd < 16:
                        sc_a = jnp.where(lane, score, sc_a)
                        ix_a = jnp.where(lane, gidx_splat, ix_a)
                    else:
                        sc_b = jnp.where(lane, score, sc_b)
                        ix_b = jnp.where(lane, gidx_splat, ix_b)
                    rv0 = jnp.where(c0 & (ri0 == gidx_splat), NEG_INF, rv0)
                    rv1 = jnp.where(c1m & (ri1 == gidx_splat), NEG_INF, rv1)
                return sc_a, ix_a, sc_b, ix_b

            sc_a, ix_a, sc_b, ix_b = lax.cond(t0 + t1 > 0, slow, fast)

            ov[i, pl.ds(0, 16)] = sc_a
            ov[i, pl.ds(16, 16)] = sc_b
            oi[i, pl.ds(0, 16)] = ix_a
            oi[i, pl.ds(16, 16)] = ix_b

        st_a0 = phase_a(jnp.int32(0), gbuf_a, sem_a)
        st_b0 = phase_a(jnp.int32(1), gbuf_b, sem_b)

        def pipe_body(t, carry):
            st_a, st_b = carry
            i = 2 * t
            process(i, gbuf_a, sem_a, st_a)
            st_a = phase_a(i + 2, gbuf_a, sem_a)
            process(i + 1, gbuf_b, sem_b, st_b)
            st_b = phase_a(i + 3, gbuf_b, sem_b)
            return st_a, st_b

        st_a, st_b = lax.fori_loop(0, rows_per_w // 2 - 1, pipe_body,
                                   (st_a0, st_b0))
        process(jnp.int32(rows_per_w - 2), gbuf_a, sem_a, st_a)
        process(jnp.int32(rows_per_w - 1), gbuf_b, sem_b, st_b)

        pltpu.sync_copy(ov, osc_hbm.at[pl.ds(base, rows_per_w)])
        pltpu.sync_copy(oi, oix_hbm.at[pl.ds(base, rows_per_w)])

    return body(sim2d, mmax, m2, s2)


@jax.jit
def kernel(queries, db):
    q_n, _ = queries.shape
    n, _ = db.shape
    k = min(K_TOP, n)
    n_chunks = n // CHUNK

    sim3d, m2, s2, mmax = _tc_stage(queries, db)
    sim2d = jnp.reshape(sim3d, (q_n * n_chunks, CHUNK))
    oscores, oinds = _sc_stage(sim2d, mmax, m2, s2,
                               q_n=q_n, n_chunks=n_chunks, k=k)

    rows = jnp.arange(q_n * k, dtype=jnp.int32) // k
    return rows, oinds[:, :k].reshape(-1), oscores[:, :k].reshape(-1)


# speedup vs baseline: 1.5786x; 1.0428x over previous
import functools

import jax
import jax.numpy as jnp
from jax import lax
from jax.experimental import pallas as pl
from jax.experimental.pallas import tpu as pltpu
from jax.experimental.pallas import tpu_sc as plsc

K_TOP = 20
CHUNK = 128
NEG_INF = float("-inf")


def _tc_kernel(q_ref, db_ref, sim_ref, m2_ref, s2_ref, m3_ref,
               ms_scr, ss_scr, *, nb, num_n):
    j = pl.program_id(1)
    tile = lax.dot_general(
        q_ref[...], db_ref[...],
        dimension_numbers=(((1,), (1,)), ((), ())),
        preferred_element_type=jnp.float32,
    )
    qb = tile.shape[0]
    nch = nb // CHUNK

    cms = []
    for c in range(nch):
        piece = tile[:, c * CHUNK:(c + 1) * CHUNK]
        sim_ref[:, c, :] = piece
        cms.append(jnp.max(piece, axis=1, keepdims=True))
    cm = jnp.concatenate(cms, axis=1)
    pad = jnp.full((qb, 128 - nch), NEG_INF, jnp.float32)
    m3_ref[...] = jnp.concatenate([cm, pad], axis=1).reshape(1, qb, 128)

    lm = jnp.max(cm, axis=1, keepdims=True)
    m_old = jnp.where(j == 0, -jnp.inf, ms_scr[...])
    s_old = jnp.where(j == 0, 0.0, ss_scr[...])
    m_new = jnp.maximum(m_old, lm)
    s_new = (s_old * jnp.exp(m_old - m_new)
             + jnp.sum(jnp.exp(tile - m_new), axis=1, keepdims=True))
    ms_scr[...] = m_new
    ss_scr[...] = s_new

    @pl.when(j == num_n - 1)
    def _():
        m2_ref[...] = jnp.broadcast_to(m_new, (qb, 128))
        s2_ref[...] = jnp.broadcast_to(s_new, (qb, 128))


def _tc_stage(queries, db):
    q_n, d = queries.shape
    n, _ = db.shape
    qb = 128
    nb = 2048
    num_q = q_n // qb
    num_n = n // nb
    n_chunks = n // CHUNK

    return pl.pallas_call(
        functools.partial(_tc_kernel, nb=nb, num_n=num_n),
        grid=(num_q, num_n),
        in_specs=[
            pl.BlockSpec((qb, d), lambda i, j: (i, 0)),
            pl.BlockSpec((nb, d), lambda i, j: (j, 0)),
        ],
        out_specs=[
            pl.BlockSpec((qb, nb // CHUNK, CHUNK), lambda i, j: (i, j, 0)),
            pl.BlockSpec((qb, 128), lambda i, j: (i, 0)),
            pl.BlockSpec((qb, 128), lambda i, j: (i, 0)),
            pl.BlockSpec((1, qb, 128), lambda i, j: (j, i, 0)),
        ],
        out_shape=[
            jax.ShapeDtypeStruct((q_n, n_chunks, CHUNK), jnp.float32),
            jax.ShapeDtypeStruct((q_n, 128), jnp.float32),
            jax.ShapeDtypeStruct((q_n, 128), jnp.float32),
            jax.ShapeDtypeStruct((num_n, q_n, 128), jnp.float32),
        ],
        scratch_shapes=[
            pltpu.VMEM((qb, 1), jnp.float32),
            pltpu.VMEM((qb, 1), jnp.float32),
        ],
        compiler_params=pltpu.CompilerParams(
            dimension_semantics=("arbitrary", "arbitrary"),
        ),
    )(queries, db)


def _cmpsel(av, ai, bv, bi):
    ge = av >= bv
    hv = jnp.where(ge, av, bv)
    hi = jnp.where(ge, ai, bi)
    lv = jnp.where(ge, bv, av)
    li = jnp.where(ge, bi, ai)
    return hv, hi, lv, li


def _rev(x):
    return lax.rev(x, (0,))


def _sort16(v, i):
    return plsc.sort_key_val(v, i, descending=True)


def _sort2_full(av, ai, bv, bi):
    rbv, rbi = _rev(bv), _rev(bi)
    hv, hi, lv, li = _cmpsel(av, ai, rbv, rbi)
    r0 = _sort16(hv, hi)
    r1 = _sort16(lv, li)
    return r0[0], r0[1], r1[0], r1[1]


def _merge32(av0, ai0, av1, ai1, bv0, bi0, bv1, bi1):
    h0v, h0i, _, _ = _cmpsel(av0, ai0, _rev(bv1), _rev(bi1))
    h1v, h1i, _, _ = _cmpsel(av1, ai1, _rev(bv0), _rev(bi0))
    ev, ei, fv, fi = _cmpsel(h0v, h0i, h1v, h1i)
    r0 = _sort16(ev, ei)
    r1 = _sort16(fv, fi)
    return r0[0], r0[1], r1[0], r1[1]


def _sc_stage(sim2d, mmax, m2, s2, *, q_n, n_chunks, k):
    info = plsc.get_sparse_core_info()
    nw = info.num_cores * info.num_subcores
    rows_per_w = q_n // nw
    n_groups = n_chunks // 16
    mesh = plsc.VectorSubcoreMesh(core_axis_name="c", subcore_axis_name="s")

    @functools.partial(
        pl.kernel,
        mesh=mesh,
        out_type=[
            jax.ShapeDtypeStruct((q_n, 32), jnp.float32),
            jax.ShapeDtypeStruct((q_n, 32), jnp.int32),
        ],
        scratch_types=[
            pltpu.VMEM((n_groups * rows_per_w, 128), jnp.float32),
            pltpu.VMEM((rows_per_w, 128), jnp.float32),
            pltpu.VMEM((rows_per_w, 128), jnp.float32),
            pltpu.VMEM((32, CHUNK), jnp.float32),
            pltpu.VMEM((32, CHUNK), jnp.float32),
            pltpu.VMEM((rows_per_w, 32), jnp.float32),
            pltpu.VMEM((rows_per_w, 32), jnp.int32),
            pltpu.SemaphoreType.DMA,
            pltpu.SemaphoreType.DMA,
        ],
        compiler_params=pltpu.CompilerParams(needs_layout_passes=False),
    )
    def body(sim_hbm, mm_hbm, m2_hbm, s2_hbm, osc_hbm, oix_hbm,
             mv, msv, ssv, gbuf_a, gbuf_b, ov, oi, sem_a, sem_b):
        wid = lax.axis_index("s") * info.num_cores + lax.axis_index("c")
        base = wid * rows_per_w
        for g in range(n_groups):
            pltpu.sync_copy(mm_hbm.at[pl.ds(g * q_n + base, rows_per_w)],
                            mv.at[pl.ds(g * rows_per_w, rows_per_w)])
        pltpu.sync_copy(m2_hbm.at[pl.ds(base, rows_per_w)], msv)
        pltpu.sync_copy(s2_hbm.at[pl.ds(base, rows_per_w)], ssv)

        iota = lax.iota(jnp.int32, 16)
        neg_inf_v = jnp.full((16,), NEG_INF, jnp.float32)
        zero_i = jnp.zeros((16,), jnp.int32)

        def phase_a(i, gbuf, sem):
            lists = []
            for g in range(n_groups):
                v = mv[g * rows_per_w + i, pl.ds(0, 16)]
                ids = iota + (g * 16)
                lists.append(_sort16(v, ids))
            l32 = []
            for p in range(n_groups // 2):
                a, b = lists[2 * p], lists[2 * p + 1]
                l32.append(_sort2_full(a[0], a[1], b[0], b[1]))
            while len(l32) > 1:
                nxt = []
                for p in range(len(l32) // 2):
                    nxt.append(_merge32(*l32[2 * p], *l32[2 * p + 1]))
                l32 = nxt
            tv0, ti0, tv1, ti1 = l32[0]

            t_splat = jnp.take(tv1, jnp.full((16,), k - 17, jnp.int32))
            cnt = (16 + plsc.all_reduce_population_count(tv1 >= t_splat)[0])

            row_off = (base + i) * n_chunks
            pltpu.async_copy(sim_hbm.at[row_off + ti0],
                             gbuf.at[pl.ds(0, 16)], sem)
            pltpu.async_copy(sim_hbm.at[row_off + ti1],
                             gbuf.at[pl.ds(16, 16)], sem)
            return ti0, ti1, t_splat, cnt

        def process(i, gbuf, sem, st):
            ti0, ti1, t_splat, cnt = st
            m_splat = msv[i, pl.ds(0, 16)]
            s_splat = ssv[i, pl.ds(0, 16)]

            pltpu.make_async_copy(sim_hbm.at[pl.ds(0, 32)], gbuf, sem).wait()

            def chunk_body(q, carry):
                rv0, ri0, rv1, ri1 = carry
                use0 = jnp.full((16,), q < 16, jnp.bool_)
                ti = jnp.where(use0, ti0, ti1)
                cid = jnp.take(ti, jnp.full((16,), q % 16, jnp.int32))
                colbase = cid * CHUNK
                for jj in range(CHUNK // 16):
                    v = gbuf[q, pl.ds(jj * 16, 16)]
                    gidx = colbase + (jj * 16) + iota
                    mask = v >= t_splat
                    pc = plsc.all_reduce_population_count(mask)[0]

                    def do_merge(rv0=rv0, ri0=ri0, rv1=rv1, ri1=ri1,
                                 v=v, gidx=gidx, mask=mask):
                        sv, si = _sort16(jnp.where(mask, v, NEG_INF), gidx)
                        return _merge32(rv0, ri0, rv1, ri1,
                                        sv, si, neg_inf_v, zero_i)

                    def no_merge(rv0=rv0, ri0=ri0, rv1=rv1, ri1=ri1):
                        return rv0, ri0, rv1, ri1

                    rv0, ri0, rv1, ri1 = lax.cond(pc > 0, do_merge, no_merge)
                return rv0, ri0, rv1, ri1

            rv0, ri0, rv1, ri1 = lax.fori_loop(
                0, cnt, chunk_body, (neg_inf_v, zero_i, neg_inf_v, zero_i))

            sh0 = jnp.where(iota == 15,
                            jnp.take(rv1, jnp.full((16,), 0, jnp.int32)),
                            jnp.take(rv0, jnp.minimum(iota + 1, 15)))
            sh1 = jnp.take(rv1, jnp.minimum(iota + 1, 15))
            t0 = plsc.all_reduce_population_count(rv0 == sh0)[0]
            t1 = plsc.all_reduce_population_count(
                (rv1 == sh1) & (iota < 5))[0]

            def fast(rv0=rv0, ri0=ri0, rv1=rv1, ri1=ri1):
                sc0 = jnp.exp(rv0 - m_splat) / s_splat
                sc1 = jnp.exp(rv1 - m_splat) / s_splat
                return sc0, ri0, sc1, ri1

            def slow(rv0=rv0, ri0=ri0, rv1=rv1, ri1=ri1):
                big = jnp.int32(2 ** 30)
                sc_a = neg_inf_v
                sc_b = neg_inf_v
                ix_a = zero_i
                ix_b = zero_i
                for rnd in range(k):
                    gm = jnp.max(jnp.maximum(rv0, rv1))
                    gm_splat = jnp.full((16,), gm, jnp.float32)
                    c0 = rv0 == gm_splat
                    c1m = rv1 == gm_splat
                    gidx = jnp.min(jnp.minimum(jnp.where(c0, ri0, big),
                                               jnp.where(c1m, ri1, big)))
                    gidx_splat = jnp.full((16,), gidx, jnp.int32)
                    score = jnp.exp(gm_splat - m_splat) / s_splat
                    lane = iota == (rnd % 16)
                    if rnd < 16:
                        sc_a = jnp.where(lane, score, sc_a)
                        ix_a = jnp.where(lane, gidx_splat, ix_a)
                    else:
                        sc_b = jnp.where(lane, score, sc_b)
                        ix_b = jnp.where(lane, gidx_splat, ix_b)
                    rv0 = jnp.where(c0 & (ri0 == gidx_splat), NEG_INF, rv0)
                    rv1 = jnp.where(c1m & (ri1 == gidx_splat), NEG_INF, rv1)
                return sc_a, ix_a, sc_b, ix_b

            sc_a, ix_a, sc_b, ix_b = lax.cond(t0 + t1 > 0, slow, fast)

            ov[i, pl.ds(0, 16)] = sc_a
            ov[i, pl.ds(16, 16)] = sc_b
            oi[i, pl.ds(0, 16)] = ix_a
            oi[i, pl.ds(16, 16)] = ix_b

        st_a0 = phase_a(jnp.int32(0), gbuf_a, sem_a)
        st_b0 = phase_a(jnp.int32(1), gbuf_b, sem_b)

        def pipe_body(t, carry):
            st_a, st_b = carry
            i = 2 * t
            process(i, gbuf_a, sem_a, st_a)
            st_a = phase_a(i + 2, gbuf_a, sem_a)
            process(i + 1, gbuf_b, sem_b, st_b)
            st_b = phase_a(i + 3, gbuf_b, sem_b)
            return st_a, st_b

        st_a, st_b = lax.fori_loop(0, rows_per_w // 2 - 1, pipe_body,
                                   (st_a0, st_b0))
        process(jnp.int32(rows_per_w - 2), gbuf_a, sem_a, st_a)
        process(jnp.int32(rows_per_w - 1), gbuf_b, sem_b, st_b)

        pltpu.sync_copy(ov, osc_hbm.at[pl.ds(base, rows_per_w)])
        pltpu.sync_copy(oi, oix_hbm.at[pl.ds(base, rows_per_w)])

    return body(sim2d, mmax, m2, s2)


@jax.jit
def kernel(queries, db):
    q_n, _ = queries.shape
    n, _ = db.shape
    k = min(K_TOP, n)
    n_chunks = n // CHUNK

    sim3d, m2, s2, m3 = _tc_stage(queries, db)
    sim2d = jnp.reshape(sim3d, (q_n * n_chunks, CHUNK))
    mmax = jnp.reshape(m3, (m3.shape[0] * q_n, 128))
    oscores, oinds = _sc_stage(sim2d, mmax, m2, s2,
                               q_n=q_n, n_chunks=n_chunks, k=k)

    rows = jnp.arange(q_n * k, dtype=jnp.int32) // k
    return rows, oinds[:, :k].reshape(-1), oscores[:, :k].reshape(-1)


# speedup vs baseline: 1.7486x; 1.1077x over previous
import functools

import jax
import jax.numpy as jnp
from jax import lax
from jax.experimental import pallas as pl
from jax.experimental.pallas import tpu as pltpu
from jax.experimental.pallas import tpu_sc as plsc

K_TOP = 20
CHUNK = 128
NEG_INF = float("-inf")


def _tc_kernel(q_ref, db_ref, sim_ref, m2_ref, s2_ref, m3_ref,
               ms_scr, ss_scr, *, nb, num_n):
    j = pl.program_id(0)
    i = pl.program_id(1)
    tile = lax.dot_general(
        q_ref[...], db_ref[...],
        dimension_numbers=(((1,), (1,)), ((), ())),
        preferred_element_type=jnp.float32,
    )
    qb = tile.shape[0]
    nch = nb // CHUNK

    cms = []
    for c in range(nch):
        piece = tile[:, c * CHUNK:(c + 1) * CHUNK]
        sim_ref[:, c, :] = piece
        cms.append(jnp.max(piece, axis=1, keepdims=True))
    cm = jnp.concatenate(cms, axis=1)
    pad = jnp.full((qb, 128 - nch), NEG_INF, jnp.float32)
    m3_ref[...] = jnp.concatenate([cm, pad], axis=1).reshape(1, qb, 128)

    lm = jnp.max(cm, axis=1, keepdims=True)
    m_old = jnp.where(j == 0, -jnp.inf, ms_scr[i])
    s_old = jnp.where(j == 0, 0.0, ss_scr[i])
    m_new = jnp.maximum(m_old, lm)
    s_new = (s_old * jnp.exp(m_old - m_new)
             + jnp.sum(jnp.exp(tile - m_new), axis=1, keepdims=True))
    ms_scr[i] = m_new
    ss_scr[i] = s_new

    @pl.when(j == num_n - 1)
    def _():
        m2_ref[...] = jnp.broadcast_to(m_new, (qb, 128))
        s2_ref[...] = jnp.broadcast_to(s_new, (qb, 128))


def _tc_stage(queries, db):
    q_n, d = queries.shape
    n, _ = db.shape
    qb = 128
    nb = 2048
    num_q = q_n // qb
    num_n = n // nb
    n_chunks = n // CHUNK

    return pl.pallas_call(
        functools.partial(_tc_kernel, nb=nb, num_n=num_n),
        grid=(num_n, num_q),
        in_specs=[
            pl.BlockSpec((qb, d), lambda j, i: (i, 0)),
            pl.BlockSpec((nb, d), lambda j, i: (j, 0)),
        ],
        out_specs=[
            pl.BlockSpec((qb, nb // CHUNK, CHUNK), lambda j, i: (i, j, 0)),
            pl.BlockSpec((qb, 128), lambda j, i: (i, 0)),
            pl.BlockSpec((qb, 128), lambda j, i: (i, 0)),
            pl.BlockSpec((1, qb, 128), lambda j, i: (j, i, 0)),
        ],
        out_shape=[
            jax.ShapeDtypeStruct((q_n, n_chunks, CHUNK), jnp.float32),
            jax.ShapeDtypeStruct((q_n, 128), jnp.float32),
            jax.ShapeDtypeStruct((q_n, 128), jnp.float32),
            jax.ShapeDtypeStruct((num_n, q_n, 128), jnp.float32),
        ],
        scratch_shapes=[
            pltpu.VMEM((num_q, qb, 1), jnp.float32),
            pltpu.VMEM((num_q, qb, 1), jnp.float32),
        ],
        compiler_params=pltpu.CompilerParams(
            dimension_semantics=("arbitrary", "arbitrary"),
        ),
    )(queries, db)


def _cmpsel(av, ai, bv, bi):
    ge = av >= bv
    hv = jnp.where(ge, av, bv)
    hi = jnp.where(ge, ai, bi)
    lv = jnp.where(ge, bv, av)
    li = jnp.where(ge, bi, ai)
    return hv, hi, lv, li


def _rev(x):
    return lax.rev(x, (0,))


def _sort16(v, i):
    return plsc.sort_key_val(v, i, descending=True)


def _sort2_full(av, ai, bv, bi):
    rbv, rbi = _rev(bv), _rev(bi)
    hv, hi, lv, li = _cmpsel(av, ai, rbv, rbi)
    r0 = _sort16(hv, hi)
    r1 = _sort16(lv, li)
    return r0[0], r0[1], r1[0], r1[1]


def _merge32(av0, ai0, av1, ai1, bv0, bi0, bv1, bi1):
    h0v, h0i, _, _ = _cmpsel(av0, ai0, _rev(bv1), _rev(bi1))
    h1v, h1i, _, _ = _cmpsel(av1, ai1, _rev(bv0), _rev(bi0))
    ev, ei, fv, fi = _cmpsel(h0v, h0i, h1v, h1i)
    r0 = _sort16(ev, ei)
    r1 = _sort16(fv, fi)
    return r0[0], r0[1], r1[0], r1[1]


def _sc_stage(sim2d, mmax, m2, s2, *, q_n, n_chunks, k):
    info = plsc.get_sparse_core_info()
    nw = info.num_cores * info.num_subcores
    rows_per_w = q_n // nw
    n_groups = n_chunks // 16
    mesh = plsc.VectorSubcoreMesh(core_axis_name="c", subcore_axis_name="s")

    @functools.partial(
        pl.kernel,
        mesh=mesh,
        out_type=[
            jax.ShapeDtypeStruct((q_n, 32), jnp.float32),
            jax.ShapeDtypeStruct((q_n, 32), jnp.int32),
        ],
        scratch_types=[
            pltpu.VMEM((n_groups * rows_per_w, 128), jnp.float32),
            pltpu.VMEM((rows_per_w, 128), jnp.float32),
            pltpu.VMEM((rows_per_w, 128), jnp.float32),
            pltpu.VMEM((32, CHUNK), jnp.float32),
            pltpu.VMEM((32, CHUNK), jnp.float32),
            pltpu.VMEM((rows_per_w, 32), jnp.float32),
            pltpu.VMEM((rows_per_w, 32), jnp.int32),
            pltpu.SemaphoreType.DMA,
            pltpu.SemaphoreType.DMA,
        ],
        compiler_params=pltpu.CompilerParams(needs_layout_passes=False),
    )
    def body(sim_hbm, mm_hbm, m2_hbm, s2_hbm, osc_hbm, oix_hbm,
             mv, msv, ssv, gbuf_a, gbuf_b, ov, oi, sem_a, sem_b):
        wid = lax.axis_index("s") * info.num_cores + lax.axis_index("c")
        base = wid * rows_per_w
        for g in range(n_groups):
            pltpu.sync_copy(mm_hbm.at[pl.ds(g * q_n + base, rows_per_w)],
                            mv.at[pl.ds(g * rows_per_w, rows_per_w)])
        pltpu.sync_copy(m2_hbm.at[pl.ds(base, rows_per_w)], msv)
        pltpu.sync_copy(s2_hbm.at[pl.ds(base, rows_per_w)], ssv)

        iota = lax.iota(jnp.int32, 16)
        neg_inf_v = jnp.full((16,), NEG_INF, jnp.float32)
        zero_i = jnp.zeros((16,), jnp.int32)

        def phase_a(i, gbuf, sem):
            lists = []
            for g in range(n_groups):
                v = mv[g * rows_per_w + i, pl.ds(0, 16)]
                ids = iota + (g * 16)
                lists.append(_sort16(v, ids))
            l32 = []
            for p in range(n_groups // 2):
                a, b = lists[2 * p], lists[2 * p + 1]
                l32.append(_sort2_full(a[0], a[1], b[0], b[1]))
            while len(l32) > 1:
                nxt = []
                for p in range(len(l32) // 2):
                    nxt.append(_merge32(*l32[2 * p], *l32[2 * p + 1]))
                l32 = nxt
            tv0, ti0, tv1, ti1 = l32[0]

            t_splat = jnp.take(tv1, jnp.full((16,), k - 17, jnp.int32))
            cnt = (16 + plsc.all_reduce_population_count(tv1 >= t_splat)[0])

            row_off = (base + i) * n_chunks
            pltpu.async_copy(sim_hbm.at[row_off + ti0],
                             gbuf.at[pl.ds(0, 16)], sem)
            pltpu.async_copy(sim_hbm.at[row_off + ti1],
                             gbuf.at[pl.ds(16, 16)], sem)
            return ti0, ti1, t_splat, cnt

        def process(i, gbuf, sem, st):
            ti0, ti1, t_splat, cnt = st
            m_splat = msv[i, pl.ds(0, 16)]
            s_splat = ssv[i, pl.ds(0, 16)]

            pltpu.make_async_copy(sim_hbm.at[pl.ds(0, 32)], gbuf, sem).wait()

            def chunk_body(q, carry):
                rv0, ri0, rv1, ri1 = carry
                use0 = jnp.full((16,), q < 16, jnp.bool_)
                ti = jnp.where(use0, ti0, ti1)
                cid = jnp.take(ti, jnp.full((16,), q % 16, jnp.int32))
                colbase = cid * CHUNK
                for jj in range(CHUNK // 16):
                    v = gbuf[q, pl.ds(jj * 16, 16)]
                    gidx = colbase + (jj * 16) + iota
                    mask = v >= t_splat
                    pc = plsc.all_reduce_population_count(mask)[0]

                    def do_merge(rv0=rv0, ri0=ri0, rv1=rv1, ri1=ri1,
                                 v=v, gidx=gidx, mask=mask):
                        sv, si = _sort16(jnp.where(mask, v, NEG_INF), gidx)
                        return _merge32(rv0, ri0, rv1, ri1,
                                        sv, si, neg_inf_v, zero_i)

                    def no_merge(rv0=rv0, ri0=ri0, rv1=rv1, ri1=ri1):
                        return rv0, ri0, rv1, ri1

                    rv0, ri0, rv1, ri1 = lax.cond(pc > 0, do_merge, no_merge)
                return rv0, ri0, rv1, ri1

            rv0, ri0, rv1, ri1 = lax.fori_loop(
                0, cnt, chunk_body, (neg_inf_v, zero_i, neg_inf_v, zero_i))

            sh0 = jnp.where(iota == 15,
                            jnp.take(rv1, jnp.full((16,), 0, jnp.int32)),
                            jnp.take(rv0, jnp.minimum(iota + 1, 15)))
            sh1 = jnp.take(rv1, jnp.minimum(iota + 1, 15))
            t0 = plsc.all_reduce_population_count(rv0 == sh0)[0]
            t1 = plsc.all_reduce_population_count(
                (rv1 == sh1) & (iota < 5))[0]

            def fast(rv0=rv0, ri0=ri0, rv1=rv1, ri1=ri1):
                sc0 = jnp.exp(rv0 - m_splat) / s_splat
                sc1 = jnp.exp(rv1 - m_splat) / s_splat
                return sc0, ri0, sc1, ri1

            def slow(rv0=rv0, ri0=ri0, rv1=rv1, ri1=ri1):
                big = jnp.int32(2 ** 30)
                sc_a = neg_inf_v
                sc_b = neg_inf_v
                ix_a = zero_i
                ix_b = zero_i
                for rnd in range(k):
                    gm = jnp.max(jnp.maximum(rv0, rv1))
                    gm_splat = jnp.full((16,), gm, jnp.float32)
                    c0 = rv0 == gm_splat
                    c1m = rv1 == gm_splat
                    gidx = jnp.min(jnp.minimum(jnp.where(c0, ri0, big),
                                               jnp.where(c1m, ri1, big)))
                    gidx_splat = jnp.full((16,), gidx, jnp.int32)
                    score = jnp.exp(gm_splat - m_splat) / s_splat
                    lane = iota == (rnd % 16)
                    if rnd < 16:
                        sc_a = jnp.where(lane, score, sc_a)
                        ix_a = jnp.where(lane, gidx_splat, ix_a)
                    else:
                        sc_b = jnp.where(lane, score, sc_b)
                        ix_b = jnp.where(lane, gidx_splat, ix_b)
                    rv0 = jnp.where(c0 & (ri0 == gidx_splat), NEG_INF, rv0)
                    rv1 = jnp.where(c1m & (ri1 == gidx_splat), NEG_INF, rv1)
                return sc_a, ix_a, sc_b, ix_b

            sc_a, ix_a, sc_b, ix_b = lax.cond(t0 + t1 > 0, slow, fast)

            ov[i, pl.ds(0, 16)] = sc_a
            ov[i, pl.ds(16, 16)] = sc_b
            oi[i, pl.ds(0, 16)] = ix_a
            oi[i, pl.ds(16, 16)] = ix_b

        st_a0 = phase_a(jnp.int32(0), gbuf_a, sem_a)
        st_b0 = phase_a(jnp.int32(1), gbuf_b, sem_b)

        def pipe_body(t, carry):
            st_a, st_b = carry
            i = 2 * t
            process(i, gbuf_a, sem_a, st_a)
            st_a = phase_a(i + 2, gbuf_a, sem_a)
            process(i + 1, gbuf_b, sem_b, st_b)
            st_b = phase_a(i + 3, gbuf_b, sem_b)
            return st_a, st_b

        st_a, st_b = lax.fori_loop(0, rows_per_w // 2 - 1, pipe_body,
                                   (st_a0, st_b0))
        process(jnp.int32(rows_per_w - 2), gbuf_a, sem_a, st_a)
        process(jnp.int32(rows_per_w - 1), gbuf_b, sem_b, st_b)

        pltpu.sync_copy(ov, osc_hbm.at[pl.ds(base, rows_per_w)])
        pltpu.sync_copy(oi, oix_hbm.at[pl.ds(base, rows_per_w)])

    return body(sim2d, mmax, m2, s2)


@jax.jit
def kernel(queries, db):
    q_n, _ = queries.shape
    n, _ = db.shape
    k = min(K_TOP, n)
    n_chunks = n // CHUNK

    sim3d, m2, s2, m3 = _tc_stage(queries, db)
    sim2d = jnp.reshape(sim3d, (q_n * n_chunks, CHUNK))
    mmax = jnp.reshape(m3, (m3.shape[0] * q_n, 128))
    oscores, oinds = _sc_stage(sim2d, mmax, m2, s2,
                               q_n=q_n, n_chunks=n_chunks, k=k)

    rows = jnp.arange(q_n * k, dtype=jnp.int32) // k
    return rows, oinds[:, :k].reshape(-1), oscores[:, :k].reshape(-1)


# speedup vs baseline: 1.8297x; 1.0464x over previous
import functools

import jax
import jax.numpy as jnp
from jax import lax
from jax.experimental import pallas as pl
from jax.experimental.pallas import tpu as pltpu
from jax.experimental.pallas import tpu_sc as plsc

K_TOP = 20
CHUNK = 128
NEG_INF = float("-inf")


def _tc_kernel(q_ref, db_ref, sim_ref, m2_ref, s2_ref, m3_ref,
               ms_scr, ss_scr, *, nb, num_n):
    j = pl.program_id(0)
    i = pl.program_id(1)
    tile = lax.dot_general(
        q_ref[...], db_ref[...],
        dimension_numbers=(((1,), (1,)), ((), ())),
        preferred_element_type=jnp.float32,
    )
    qb = tile.shape[0]
    nch = nb // CHUNK

    cms = []
    for c in range(nch):
        piece = tile[:, c * CHUNK:(c + 1) * CHUNK]
        sim_ref[:, c, :] = piece
        cms.append(jnp.max(piece, axis=1, keepdims=True))
    cm = jnp.concatenate(cms, axis=1)
    pad = jnp.full((qb, 128 - nch), NEG_INF, jnp.float32)
    m3_ref[...] = jnp.concatenate([cm, pad], axis=1).reshape(1, qb, 128)

    lm = jnp.max(cm, axis=1, keepdims=True)
    m_old = jnp.where(j == 0, -jnp.inf, ms_scr[i])
    s_old = jnp.where(j == 0, 0.0, ss_scr[i])
    m_new = jnp.maximum(m_old, lm)
    s_new = (s_old * jnp.exp(m_old - m_new)
             + jnp.sum(jnp.exp(tile - m_new), axis=1, keepdims=True))
    ms_scr[i] = m_new
    ss_scr[i] = s_new

    @pl.when(j == num_n - 1)
    def _():
        m2_ref[...] = jnp.broadcast_to(m_new, (qb, 128))
        s2_ref[...] = jnp.broadcast_to(s_new, (qb, 128))


def _tc_stage(queries, db):
    q_n, d = queries.shape
    n, _ = db.shape
    qb = 128
    nb = 2048
    num_q = q_n // qb
    num_n = n // nb
    n_chunks = n // CHUNK

    return pl.pallas_call(
        functools.partial(_tc_kernel, nb=nb, num_n=num_n),
        grid=(num_n, num_q),
        in_specs=[
            pl.BlockSpec((qb, d), lambda j, i: (i, 0)),
            pl.BlockSpec((nb, d), lambda j, i: (j, 0)),
        ],
        out_specs=[
            pl.BlockSpec((qb, nb // CHUNK, CHUNK), lambda j, i: (i, j, 0)),
            pl.BlockSpec((qb, 128), lambda j, i: (i, 0)),
            pl.BlockSpec((qb, 128), lambda j, i: (i, 0)),
            pl.BlockSpec((1, qb, 128), lambda j, i: (j, i, 0)),
        ],
        out_shape=[
            jax.ShapeDtypeStruct((q_n, n_chunks, CHUNK), jnp.float32),
            jax.ShapeDtypeStruct((q_n, 128), jnp.float32),
            jax.ShapeDtypeStruct((q_n, 128), jnp.float32),
            jax.ShapeDtypeStruct((num_n, q_n, 128), jnp.float32),
        ],
        scratch_shapes=[
            pltpu.VMEM((num_q, qb, 1), jnp.float32),
            pltpu.VMEM((num_q, qb, 1), jnp.float32),
        ],
        compiler_params=pltpu.CompilerParams(
            dimension_semantics=("arbitrary", "arbitrary"),
        ),
    )(queries, db)


def _cmpsel(av, ai, bv, bi):
    ge = av >= bv
    hv = jnp.where(ge, av, bv)
    hi = jnp.where(ge, ai, bi)
    lv = jnp.where(ge, bv, av)
    li = jnp.where(ge, bi, ai)
    return hv, hi, lv, li


def _rev(x):
    return lax.rev(x, (0,))


def _sort16(v, i):
    return plsc.sort_key_val(v, i, descending=True)


def _sort2_full(av, ai, bv, bi):
    rbv, rbi = _rev(bv), _rev(bi)
    hv, hi, lv, li = _cmpsel(av, ai, rbv, rbi)
    r0 = _sort16(hv, hi)
    r1 = _sort16(lv, li)
    return r0[0], r0[1], r1[0], r1[1]


def _merge32(av0, ai0, av1, ai1, bv0, bi0, bv1, bi1):
    h0v, h0i, _, _ = _cmpsel(av0, ai0, _rev(bv1), _rev(bi1))
    h1v, h1i, _, _ = _cmpsel(av1, ai1, _rev(bv0), _rev(bi0))
    ev, ei, fv, fi = _cmpsel(h0v, h0i, h1v, h1i)
    r0 = _sort16(ev, ei)
    r1 = _sort16(fv, fi)
    return r0[0], r0[1], r1[0], r1[1]


def _sc_stage(sim2d, mmax, m2, s2, *, q_n, n_chunks, k):
    info = plsc.get_sparse_core_info()
    nw = info.num_cores * info.num_subcores
    rows_per_w = q_n // nw
    n_groups = n_chunks // 16
    mesh = plsc.VectorSubcoreMesh(core_axis_name="c", subcore_axis_name="s")

    @functools.partial(
        pl.kernel,
        mesh=mesh,
        out_type=[
            jax.ShapeDtypeStruct((q_n, 32), jnp.float32),
            jax.ShapeDtypeStruct((q_n, 32), jnp.int32),
        ],
        scratch_types=[
            pltpu.VMEM((n_groups * rows_per_w, 128), jnp.float32),
            pltpu.VMEM((rows_per_w, 128), jnp.float32),
            pltpu.VMEM((rows_per_w, 128), jnp.float32),
            pltpu.VMEM((32, CHUNK), jnp.float32),
            pltpu.VMEM((32, CHUNK), jnp.float32),
            pltpu.VMEM((rows_per_w, 32), jnp.float32),
            pltpu.VMEM((rows_per_w, 32), jnp.int32),
            pltpu.SemaphoreType.DMA,
            pltpu.SemaphoreType.DMA,
        ],
        compiler_params=pltpu.CompilerParams(needs_layout_passes=False),
    )
    def body(sim_hbm, mm_hbm, m2_hbm, s2_hbm, osc_hbm, oix_hbm,
             mv, msv, ssv, gbuf_a, gbuf_b, ov, oi, sem_a, sem_b):
        wid = lax.axis_index("s") * info.num_cores + lax.axis_index("c")
        base = wid * rows_per_w
        for g in range(n_groups):
            pltpu.sync_copy(mm_hbm.at[pl.ds(g * q_n + base, rows_per_w)],
                            mv.at[pl.ds(g * rows_per_w, rows_per_w)])
        pltpu.sync_copy(m2_hbm.at[pl.ds(base, rows_per_w)], msv)
        pltpu.sync_copy(s2_hbm.at[pl.ds(base, rows_per_w)], ssv)

        iota = lax.iota(jnp.int32, 16)
        neg_inf_v = jnp.full((16,), NEG_INF, jnp.float32)
        zero_i = jnp.zeros((16,), jnp.int32)

        def phase_a(i, gbuf, sem):
            lists = []
            for g in range(n_groups):
                v = mv[g * rows_per_w + i, pl.ds(0, 16)]
                ids = iota + (g * 16)
                lists.append(_sort16(v, ids))
            l32 = []
            for p in range(n_groups // 2):
                a, b = lists[2 * p], lists[2 * p + 1]
                l32.append(_sort2_full(a[0], a[1], b[0], b[1]))
            while len(l32) > 1:
                nxt = []
                for p in range(len(l32) // 2):
                    nxt.append(_merge32(*l32[2 * p], *l32[2 * p + 1]))
                l32 = nxt
            tv0, ti0, tv1, ti1 = l32[0]

            t_splat = jnp.take(tv1, jnp.full((16,), k - 17, jnp.int32))
            cnt = (16 + plsc.all_reduce_population_count(tv1 >= t_splat)[0])

            row_off = (base + i) * n_chunks
            pltpu.async_copy(sim_hbm.at[row_off + ti0],
                             gbuf.at[pl.ds(0, 16)], sem)
            pltpu.async_copy(sim_hbm.at[row_off + ti1],
                             gbuf.at[pl.ds(16, 16)], sem)
            return ti0, ti1, t_splat, cnt

        def process(i, gbuf, sem, st):
            ti0, ti1, t_splat, cnt = st
            m_splat = msv[i, pl.ds(0, 16)]
            s_splat = ssv[i, pl.ds(0, 16)]

            pltpu.make_async_copy(sim_hbm.at[pl.ds(0, 32)], gbuf, sem).wait()

            def chunk_body(q, carry):
                rv0, ri0, rv1, ri1 = carry
                use0 = jnp.full((16,), q < 16, jnp.bool_)
                ti = jnp.where(use0, ti0, ti1)
                cid = jnp.take(ti, jnp.full((16,), q % 16, jnp.int32))
                colbase = cid * CHUNK
                for jj in range(CHUNK // 16):
                    v = gbuf[q, pl.ds(jj * 16, 16)]
                    gidx = colbase + (jj * 16) + iota
                    mask = v >= t_splat
                    pc = plsc.all_reduce_population_count(mask)[0]

                    def do_merge(rv0=rv0, ri0=ri0, rv1=rv1, ri1=ri1,
                                 v=v, gidx=gidx, mask=mask):
                        sv, si = _sort16(jnp.where(mask, v, NEG_INF), gidx)
                        return _merge32(rv0, ri0, rv1, ri1,
                                        sv, si, neg_inf_v, zero_i)

                    def no_merge(rv0=rv0, ri0=ri0, rv1=rv1, ri1=ri1):
                        return rv0, ri0, rv1, ri1

                    rv0, ri0, rv1, ri1 = lax.cond(pc > 0, do_merge, no_merge)
                return rv0, ri0, rv1, ri1

            rv0, ri0, rv1, ri1 = lax.fori_loop(
                0, cnt, chunk_body, (neg_inf_v, zero_i, neg_inf_v, zero_i))

            sh0 = jnp.where(iota == 15,
                            jnp.take(rv1, jnp.full((16,), 0, jnp.int32)),
                            jnp.take(rv0, jnp.minimum(iota + 1, 15)))
            sh1 = jnp.take(rv1, jnp.minimum(iota + 1, 15))
            t0 = plsc.all_reduce_population_count(rv0 == sh0)[0]
            t1 = plsc.all_reduce_population_count(
                (rv1 == sh1) & (iota < 5))[0]

            def fast(rv0=rv0, ri0=ri0, rv1=rv1, ri1=ri1):
                sc0 = jnp.exp(rv0 - m_splat) / s_splat
                sc1 = jnp.exp(rv1 - m_splat) / s_splat
                return sc0, ri0, sc1, ri1

            def slow(rv0=rv0, ri0=ri0, rv1=rv1, ri1=ri1):
                big = jnp.int32(2 ** 30)
                sc_a = neg_inf_v
                sc_b = neg_inf_v
                ix_a = zero_i
                ix_b = zero_i
                for rnd in range(k):
                    gm = jnp.max(jnp.maximum(rv0, rv1))
                    gm_splat = jnp.full((16,), gm, jnp.float32)
                    c0 = rv0 == gm_splat
                    c1m = rv1 == gm_splat
                    gidx = jnp.min(jnp.minimum(jnp.where(c0, ri0, big),
                                               jnp.where(c1m, ri1, big)))
                    gidx_splat = jnp.full((16,), gidx, jnp.int32)
                    score = jnp.exp(gm_splat - m_splat) / s_splat
                    lane = iota == (rnd % 16)
                    if rnd < 16:
                        sc_a = jnp.where(lane, score, sc_a)
                        ix_a = jnp.where(lane, gidx_splat, ix_a)
                    else:
                        sc_b = jnp.where(lane, score, sc_b)
                        ix_b = jnp.where(lane, gidx_splat, ix_b)
                    rv0 = jnp.where(c0 & (ri0 == gidx_splat), NEG_INF, rv0)
                    rv1 = jnp.where(c1m & (ri1 == gidx_splat), NEG_INF, rv1)
                return sc_a, ix_a, sc_b, ix_b

            sc_a, ix_a, sc_b, ix_b = lax.cond(t0 + t1 > 0, slow, fast)

            ov[i, pl.ds(0, 16)] = sc_a
            ov[i, pl.ds(16, 16)] = sc_b
            oi[i, pl.ds(0, 16)] = ix_a
            oi[i, pl.ds(16, 16)] = ix_b

        st_a0 = phase_a(jnp.int32(0), gbuf_a, sem_a)
        st_b0 = phase_a(jnp.int32(1), gbuf_b, sem_b)

        def pipe_body(t, carry):
            st_a, st_b = carry
            i = 2 * t
            process(i, gbuf_a, sem_a, st_a)
            st_a = phase_a(i + 2, gbuf_a, sem_a)
            process(i + 1, gbuf_b, sem_b, st_b)
            st_b = phase_a(i + 3, gbuf_b, sem_b)
            return st_a, st_b

        st_a, st_b = lax.fori_loop(0, rows_per_w // 2 - 1, pipe_body,
                                   (st_a0, st_b0))
        process(jnp.int32(rows_per_w - 2), gbuf_a, sem_a, st_a)
        process(jnp.int32(rows_per_w - 1), gbuf_b, sem_b, st_b)

        pltpu.sync_copy(ov, osc_hbm.at[pl.ds(base, rows_per_w)])
        pltpu.sync_copy(oi, oix_hbm.at[pl.ds(base, rows_per_w)])

    return body(sim2d, mmax, m2, s2)


@jax.jit
def kernel(queries, db):
    q_n, _ = queries.shape
    n, _ = db.shape
    k = min(K_TOP, n)
    n_chunks = n // CHUNK

    h = q_n // 2
    tc_outs = [_tc_stage(queries[p * h:(p + 1) * h], db) for p in range(2)]
    sc_outs = []
    for sim3d, m2, s2, m3 in tc_outs:
        sim2d = jnp.reshape(sim3d, (h * n_chunks, CHUNK))
        mmax = jnp.reshape(m3, (m3.shape[0] * h, 128))
        sc_outs.append(_sc_stage(sim2d, mmax, m2, s2,
                                 q_n=h, n_chunks=n_chunks, k=k))
    oscores = jnp.concatenate([o[0] for o in sc_outs], axis=0)
    oinds = jnp.concatenate([o[1] for o in sc_outs], axis=0)

    rows = jnp.arange(q_n * k, dtype=jnp.int32) // k
    return rows, oinds[:, :k].reshape(-1), oscores[:, :k].reshape(-1)


# speedup vs baseline: 2.5756x; 1.4076x over previous
import functools

import jax
import jax.numpy as jnp
from jax import lax
from jax.experimental import pallas as pl
from jax.experimental.pallas import tpu as pltpu
from jax.experimental.pallas import tpu_sc as plsc

K_TOP = 20
CHUNK = 128
NEG_INF = float("-inf")


def _tc_kernel(q_ref, db_ref, sim_ref, m2_ref, s2_ref, m3_ref,
               ms_scr, ss_scr, *, nb, num_n):
    j = pl.program_id(0)
    i = pl.program_id(1)
    tile = lax.dot_general(
        q_ref[...], db_ref[...],
        dimension_numbers=(((1,), (1,)), ((), ())),
        preferred_element_type=jnp.float32,
    )
    qb = tile.shape[0]
    nch = nb // CHUNK

    cms = []
    for c in range(nch):
        piece = tile[:, c * CHUNK:(c + 1) * CHUNK]
        sim_ref[:, c, :] = piece
        cms.append(jnp.max(piece, axis=1, keepdims=True))
    cm = jnp.concatenate(cms, axis=1)
    pad = jnp.full((qb, 128 - nch), NEG_INF, jnp.float32)
    m3_ref[...] = jnp.concatenate([cm, pad], axis=1).reshape(1, qb, 128)

    lm = jnp.max(cm, axis=1, keepdims=True)
    m_old = jnp.where(j == 0, -jnp.inf, ms_scr[i])
    s_old = jnp.where(j == 0, 0.0, ss_scr[i])
    m_new = jnp.maximum(m_old, lm)
    s_new = (s_old * jnp.exp(m_old - m_new)
             + jnp.sum(jnp.exp(tile - m_new), axis=1, keepdims=True))
    ms_scr[i] = m_new
    ss_scr[i] = s_new

    @pl.when(j == num_n - 1)
    def _():
        m2_ref[...] = jnp.broadcast_to(m_new, (qb, 128))
        s2_ref[...] = jnp.broadcast_to(s_new, (qb, 128))


def _tc_stage(queries, db):
    q_n, d = queries.shape
    n, _ = db.shape
    qb = 256
    nb = 4096
    num_q = q_n // qb
    num_n = n // nb
    n_chunks = n // CHUNK

    return pl.pallas_call(
        functools.partial(_tc_kernel, nb=nb, num_n=num_n),
        grid=(num_n, num_q),
        in_specs=[
            pl.BlockSpec((qb, d), lambda j, i: (i, 0)),
            pl.BlockSpec((nb, d), lambda j, i: (j, 0)),
        ],
        out_specs=[
            pl.BlockSpec((qb, nb // CHUNK, CHUNK), lambda j, i: (i, j, 0)),
            pl.BlockSpec((qb, 128), lambda j, i: (i, 0)),
            pl.BlockSpec((qb, 128), lambda j, i: (i, 0)),
            pl.BlockSpec((1, qb, 128), lambda j, i: (j, i, 0)),
        ],
        out_shape=[
            jax.ShapeDtypeStruct((q_n, n_chunks, CHUNK), jnp.float32),
            jax.ShapeDtypeStruct((q_n, 128), jnp.float32),
            jax.ShapeDtypeStruct((q_n, 128), jnp.float32),
            jax.ShapeDtypeStruct((num_n, q_n, 128), jnp.float32),
        ],
        scratch_shapes=[
            pltpu.VMEM((num_q, qb, 1), jnp.float32),
            pltpu.VMEM((num_q, qb, 1), jnp.float32),
        ],
        compiler_params=pltpu.CompilerParams(
            dimension_semantics=("arbitrary", "arbitrary"),
        ),
    )(queries, db)


def _cmpsel(av, ai, bv, bi):
    ge = av >= bv
    hv = jnp.where(ge, av, bv)
    hi = jnp.where(ge, ai, bi)
    lv = jnp.where(ge, bv, av)
    li = jnp.where(ge, bi, ai)
    return hv, hi, lv, li


def _rev(x):
    return lax.rev(x, (0,))


def _sort16(v, i):
    return plsc.sort_key_val(v, i, descending=True)


def _sort2_full(av, ai, bv, bi):
    rbv, rbi = _rev(bv), _rev(bi)
    hv, hi, lv, li = _cmpsel(av, ai, rbv, rbi)
    r0 = _sort16(hv, hi)
    r1 = _sort16(lv, li)
    return r0[0], r0[1], r1[0], r1[1]


def _merge32(av0, ai0, av1, ai1, bv0, bi0, bv1, bi1):
    h0v, h0i, _, _ = _cmpsel(av0, ai0, _rev(bv1), _rev(bi1))
    h1v, h1i, _, _ = _cmpsel(av1, ai1, _rev(bv0), _rev(bi0))
    ev, ei, fv, fi = _cmpsel(h0v, h0i, h1v, h1i)
    r0 = _sort16(ev, ei)
    r1 = _sort16(fv, fi)
    return r0[0], r0[1], r1[0], r1[1]


def _sc_stage(sim2d, mmax, m2, s2, *, q_n, n_chunks, n_blocks, k):
    info = plsc.get_sparse_core_info()
    nw = info.num_cores * info.num_subcores
    rows_per_w = q_n // nw
    n_groups = n_chunks // 16
    subs = n_groups // n_blocks
    mesh = plsc.VectorSubcoreMesh(core_axis_name="c", subcore_axis_name="s")

    @functools.partial(
        pl.kernel,
        mesh=mesh,
        out_type=[
            jax.ShapeDtypeStruct((q_n, 32), jnp.float32),
            jax.ShapeDtypeStruct((q_n, 32), jnp.int32),
        ],
        scratch_types=[
            pltpu.VMEM((n_blocks * rows_per_w, 128), jnp.float32),
            pltpu.VMEM((rows_per_w, 128), jnp.float32),
            pltpu.VMEM((rows_per_w, 128), jnp.float32),
            pltpu.VMEM((32, CHUNK), jnp.float32),
            pltpu.VMEM((32, CHUNK), jnp.float32),
            pltpu.VMEM((rows_per_w, 32), jnp.float32),
            pltpu.VMEM((rows_per_w, 32), jnp.int32),
            pltpu.SemaphoreType.DMA,
            pltpu.SemaphoreType.DMA,
        ],
        compiler_params=pltpu.CompilerParams(needs_layout_passes=False),
    )
    def body(sim_hbm, mm_hbm, m2_hbm, s2_hbm, osc_hbm, oix_hbm,
             mv, msv, ssv, gbuf_a, gbuf_b, ov, oi, sem_a, sem_b):
        wid = lax.axis_index("s") * info.num_cores + lax.axis_index("c")
        base = wid * rows_per_w
        for g in range(n_blocks):
            pltpu.sync_copy(mm_hbm.at[pl.ds(g * q_n + base, rows_per_w)],
                            mv.at[pl.ds(g * rows_per_w, rows_per_w)])
        pltpu.sync_copy(m2_hbm.at[pl.ds(base, rows_per_w)], msv)
        pltpu.sync_copy(s2_hbm.at[pl.ds(base, rows_per_w)], ssv)

        iota = lax.iota(jnp.int32, 16)
        neg_inf_v = jnp.full((16,), NEG_INF, jnp.float32)
        zero_i = jnp.zeros((16,), jnp.int32)

        def phase_a(i, gbuf, sem):
            lists = []
            for g in range(n_groups):
                blk, sub = g // subs, g % subs
                v = mv[blk * rows_per_w + i, pl.ds(sub * 16, 16)]
                ids = iota + (g * 16)
                lists.append(_sort16(v, ids))
            l32 = []
            for p in range(n_groups // 2):
                a, b = lists[2 * p], lists[2 * p + 1]
                l32.append(_sort2_full(a[0], a[1], b[0], b[1]))
            while len(l32) > 1:
                nxt = []
                for p in range(len(l32) // 2):
                    nxt.append(_merge32(*l32[2 * p], *l32[2 * p + 1]))
                l32 = nxt
            tv0, ti0, tv1, ti1 = l32[0]

            t_splat = jnp.take(tv1, jnp.full((16,), k - 17, jnp.int32))
            cnt = (16 + plsc.all_reduce_population_count(tv1 >= t_splat)[0])

            row_off = (base + i) * n_chunks
            pltpu.async_copy(sim_hbm.at[row_off + ti0],
                             gbuf.at[pl.ds(0, 16)], sem)
            pltpu.async_copy(sim_hbm.at[row_off + ti1],
                             gbuf.at[pl.ds(16, 16)], sem)
            return ti0, ti1, t_splat, cnt

        def process(i, gbuf, sem, st):
            ti0, ti1, t_splat, cnt = st
            m_splat = msv[i, pl.ds(0, 16)]
            s_splat = ssv[i, pl.ds(0, 16)]

            pltpu.make_async_copy(sim_hbm.at[pl.ds(0, 32)], gbuf, sem).wait()

            def chunk_body(q, carry):
                rv0, ri0, rv1, ri1 = carry
                use0 = jnp.full((16,), q < 16, jnp.bool_)
                ti = jnp.where(use0, ti0, ti1)
                cid = jnp.take(ti, jnp.full((16,), q % 16, jnp.int32))
                colbase = cid * CHUNK
                for jj in range(CHUNK // 16):
                    v = gbuf[q, pl.ds(jj * 16, 16)]
                    gidx = colbase + (jj * 16) + iota
                    mask = v >= t_splat
                    pc = plsc.all_reduce_population_count(mask)[0]

                    def do_merge(rv0=rv0, ri0=ri0, rv1=rv1, ri1=ri1,
                                 v=v, gidx=gidx, mask=mask):
                        sv, si = _sort16(jnp.where(mask, v, NEG_INF), gidx)
                        return _merge32(rv0, ri0, rv1, ri1,
                                        sv, si, neg_inf_v, zero_i)

                    def no_merge(rv0=rv0, ri0=ri0, rv1=rv1, ri1=ri1):
                        return rv0, ri0, rv1, ri1

                    rv0, ri0, rv1, ri1 = lax.cond(pc > 0, do_merge, no_merge)
                return rv0, ri0, rv1, ri1

            rv0, ri0, rv1, ri1 = lax.fori_loop(
                0, cnt, chunk_body, (neg_inf_v, zero_i, neg_inf_v, zero_i))

            sh0 = jnp.where(iota == 15,
                            jnp.take(rv1, jnp.full((16,), 0, jnp.int32)),
                            jnp.take(rv0, jnp.minimum(iota + 1, 15)))
            sh1 = jnp.take(rv1, jnp.minimum(iota + 1, 15))
            t0 = plsc.all_reduce_population_count(rv0 == sh0)[0]
            t1 = plsc.all_reduce_population_count(
                (rv1 == sh1) & (iota < 5))[0]

            def fast(rv0=rv0, ri0=ri0, rv1=rv1, ri1=ri1):
                sc0 = jnp.exp(rv0 - m_splat) / s_splat
                sc1 = jnp.exp(rv1 - m_splat) / s_splat
                return sc0, ri0, sc1, ri1

            def slow(rv0=rv0, ri0=ri0, rv1=rv1, ri1=ri1):
                big = jnp.int32(2 ** 30)
                sc_a = neg_inf_v
                sc_b = neg_inf_v
                ix_a = zero_i
                ix_b = zero_i
                for rnd in range(k):
                    gm = jnp.max(jnp.maximum(rv0, rv1))
                    gm_splat = jnp.full((16,), gm, jnp.float32)
                    c0 = rv0 == gm_splat
                    c1m = rv1 == gm_splat
                    gidx = jnp.min(jnp.minimum(jnp.where(c0, ri0, big),
                                               jnp.where(c1m, ri1, big)))
                    gidx_splat = jnp.full((16,), gidx, jnp.int32)
                    score = jnp.exp(gm_splat - m_splat) / s_splat
                    lane = iota == (rnd % 16)
                    if rnd < 16:
                        sc_a = jnp.where(lane, score, sc_a)
                        ix_a = jnp.where(lane, gidx_splat, ix_a)
                    else:
                        sc_b = jnp.where(lane, score, sc_b)
                        ix_b = jnp.where(lane, gidx_splat, ix_b)
                    rv0 = jnp.where(c0 & (ri0 == gidx_splat), NEG_INF, rv0)
                    rv1 = jnp.where(c1m & (ri1 == gidx_splat), NEG_INF, rv1)
                return sc_a, ix_a, sc_b, ix_b

            sc_a, ix_a, sc_b, ix_b = lax.cond(t0 + t1 > 0, slow, fast)

            ov[i, pl.ds(0, 16)] = sc_a
            ov[i, pl.ds(16, 16)] = sc_b
            oi[i, pl.ds(0, 16)] = ix_a
            oi[i, pl.ds(16, 16)] = ix_b

        st_a0 = phase_a(jnp.int32(0), gbuf_a, sem_a)
        st_b0 = phase_a(jnp.int32(1), gbuf_b, sem_b)

        def pipe_body(t, carry):
            st_a, st_b = carry
            i = 2 * t
            process(i, gbuf_a, sem_a, st_a)
            st_a = phase_a(i + 2, gbuf_a, sem_a)
            process(i + 1, gbuf_b, sem_b, st_b)
            st_b = phase_a(i + 3, gbuf_b, sem_b)
            return st_a, st_b

        st_a, st_b = lax.fori_loop(0, rows_per_w // 2 - 1, pipe_body,
                                   (st_a0, st_b0))
        process(jnp.int32(rows_per_w - 2), gbuf_a, sem_a, st_a)
        process(jnp.int32(rows_per_w - 1), gbuf_b, sem_b, st_b)

        pltpu.sync_copy(ov, osc_hbm.at[pl.ds(base, rows_per_w)])
        pltpu.sync_copy(oi, oix_hbm.at[pl.ds(base, rows_per_w)])

    return body(sim2d, mmax, m2, s2)


@jax.jit
def kernel(queries, db):
    q_n, _ = queries.shape
    n, _ = db.shape
    k = min(K_TOP, n)
    n_chunks = n // CHUNK

    h = q_n // 2
    tc_outs = [_tc_stage(queries[p * h:(p + 1) * h], db) for p in range(2)]
    sc_outs = []
    for sim3d, m2, s2, m3 in tc_outs:
        sim2d = jnp.reshape(sim3d, (h * n_chunks, CHUNK))
        mmax = jnp.reshape(m3, (m3.shape[0] * h, 128))
        sc_outs.append(_sc_stage(sim2d, mmax, m2, s2, q_n=h,
                                 n_chunks=n_chunks, n_blocks=m3.shape[0],
                                 k=k))
    oscores = jnp.concatenate([o[0] for o in sc_outs], axis=0)
    oinds = jnp.concatenate([o[1] for o in sc_outs], axis=0)

    rows = jnp.arange(q_n * k, dtype=jnp.int32) // k
    return rows, oinds[:, :k].reshape(-1), oscores[:, :k].reshape(-1)


# speedup vs baseline: 2.6858x; 1.0428x over previous
import functools

import jax
import jax.numpy as jnp
from jax import lax
from jax.experimental import pallas as pl
from jax.experimental.pallas import tpu as pltpu
from jax.experimental.pallas import tpu_sc as plsc

K_TOP = 20
CHUNK = 128
NEG_INF = float("-inf")


def _tc_kernel(q_ref, db_ref, sim_ref, m2_ref, s2_ref, m3_ref,
               ms_scr, ss_scr, *, nb, num_n):
    j = pl.program_id(0)
    i = pl.program_id(1)
    tile = lax.dot_general(
        q_ref[...], db_ref[...],
        dimension_numbers=(((1,), (1,)), ((), ())),
        preferred_element_type=jnp.float32,
    )
    qb = tile.shape[0]
    nch = nb // CHUNK

    cms = []
    for c in range(nch):
        piece = tile[:, c * CHUNK:(c + 1) * CHUNK]
        sim_ref[:, c, :] = piece
        cms.append(jnp.max(piece, axis=1, keepdims=True))
    cm = jnp.concatenate(cms, axis=1)
    pad = jnp.full((qb, 128 - nch), NEG_INF, jnp.float32)
    m3_ref[...] = jnp.concatenate([cm, pad], axis=1).reshape(1, qb, 128)

    lm = jnp.max(cm, axis=1, keepdims=True)
    m_old = jnp.where(j == 0, -jnp.inf, ms_scr[i])
    s_old = jnp.where(j == 0, 0.0, ss_scr[i])
    m_new = jnp.maximum(m_old, lm)
    s_new = (s_old * jnp.exp(m_old - m_new)
             + jnp.sum(jnp.exp(tile - m_new), axis=1, keepdims=True))
    ms_scr[i] = m_new
    ss_scr[i] = s_new

    @pl.when(j == num_n - 1)
    def _():
        m2_ref[...] = jnp.broadcast_to(m_new, (qb, 128))
        s2_ref[...] = jnp.broadcast_to(s_new, (qb, 128))


def _tc_stage(queries, db):
    q_n, d = queries.shape
    n, _ = db.shape
    qb = 512
    nb = 4096
    num_q = q_n // qb
    num_n = n // nb
    n_chunks = n // CHUNK

    return pl.pallas_call(
        functools.partial(_tc_kernel, nb=nb, num_n=num_n),
        grid=(num_n, num_q),
        in_specs=[
            pl.BlockSpec((qb, d), lambda j, i: (i, 0)),
            pl.BlockSpec((nb, d), lambda j, i: (j, 0)),
        ],
        out_specs=[
            pl.BlockSpec((qb, nb // CHUNK, CHUNK), lambda j, i: (i, j, 0)),
            pl.BlockSpec((qb, 128), lambda j, i: (i, 0)),
            pl.BlockSpec((qb, 128), lambda j, i: (i, 0)),
            pl.BlockSpec((1, qb, 128), lambda j, i: (j, i, 0)),
        ],
        out_shape=[
            jax.ShapeDtypeStruct((q_n, n_chunks, CHUNK), jnp.float32),
            jax.ShapeDtypeStruct((q_n, 128), jnp.float32),
            jax.ShapeDtypeStruct((q_n, 128), jnp.float32),
            jax.ShapeDtypeStruct((num_n, q_n, 128), jnp.float32),
        ],
        scratch_shapes=[
            pltpu.VMEM((num_q, qb, 1), jnp.float32),
            pltpu.VMEM((num_q, qb, 1), jnp.float32),
        ],
        compiler_params=pltpu.CompilerParams(
            dimension_semantics=("arbitrary", "arbitrary"),
        ),
    )(queries, db)


def _cmpsel(av, ai, bv, bi):
    ge = av >= bv
    hv = jnp.where(ge, av, bv)
    hi = jnp.where(ge, ai, bi)
    lv = jnp.where(ge, bv, av)
    li = jnp.where(ge, bi, ai)
    return hv, hi, lv, li


def _rev(x):
    return lax.rev(x, (0,))


def _sort16(v, i):
    return plsc.sort_key_val(v, i, descending=True)


def _sort2_full(av, ai, bv, bi):
    rbv, rbi = _rev(bv), _rev(bi)
    hv, hi, lv, li = _cmpsel(av, ai, rbv, rbi)
    r0 = _sort16(hv, hi)
    r1 = _sort16(lv, li)
    return r0[0], r0[1], r1[0], r1[1]


def _merge32(av0, ai0, av1, ai1, bv0, bi0, bv1, bi1):
    h0v, h0i, _, _ = _cmpsel(av0, ai0, _rev(bv1), _rev(bi1))
    h1v, h1i, _, _ = _cmpsel(av1, ai1, _rev(bv0), _rev(bi0))
    ev, ei, fv, fi = _cmpsel(h0v, h0i, h1v, h1i)
    r0 = _sort16(ev, ei)
    r1 = _sort16(fv, fi)
    return r0[0], r0[1], r1[0], r1[1]


def _sc_stage(sim2d, mmax, m2, s2, *, q_n, n_chunks, n_blocks, k):
    info = plsc.get_sparse_core_info()
    nw = info.num_cores * info.num_subcores
    rows_per_w = q_n // nw
    n_groups = n_chunks // 16
    subs = n_groups // n_blocks
    mesh = plsc.VectorSubcoreMesh(core_axis_name="c", subcore_axis_name="s")

    @functools.partial(
        pl.kernel,
        mesh=mesh,
        out_type=[
            jax.ShapeDtypeStruct((q_n, 32), jnp.float32),
            jax.ShapeDtypeStruct((q_n, 32), jnp.int32),
        ],
        scratch_types=[
            pltpu.VMEM((n_blocks * rows_per_w, 128), jnp.float32),
            pltpu.VMEM((rows_per_w, 128), jnp.float32),
            pltpu.VMEM((rows_per_w, 128), jnp.float32),
            pltpu.VMEM((32, CHUNK), jnp.float32),
            pltpu.VMEM((32, CHUNK), jnp.float32),
            pltpu.VMEM((rows_per_w, 32), jnp.float32),
            pltpu.VMEM((rows_per_w, 32), jnp.int32),
            pltpu.SemaphoreType.DMA,
            pltpu.SemaphoreType.DMA,
        ],
        compiler_params=pltpu.CompilerParams(needs_layout_passes=False),
    )
    def body(sim_hbm, mm_hbm, m2_hbm, s2_hbm, osc_hbm, oix_hbm,
             mv, msv, ssv, gbuf_a, gbuf_b, ov, oi, sem_a, sem_b):
        wid = lax.axis_index("s") * info.num_cores + lax.axis_index("c")
        base = wid * rows_per_w
        for g in range(n_blocks):
            pltpu.sync_copy(mm_hbm.at[pl.ds(g * q_n + base, rows_per_w)],
                            mv.at[pl.ds(g * rows_per_w, rows_per_w)])
        pltpu.sync_copy(m2_hbm.at[pl.ds(base, rows_per_w)], msv)
        pltpu.sync_copy(s2_hbm.at[pl.ds(base, rows_per_w)], ssv)

        iota = lax.iota(jnp.int32, 16)
        neg_inf_v = jnp.full((16,), NEG_INF, jnp.float32)
        zero_i = jnp.zeros((16,), jnp.int32)

        def phase_a(i, gbuf, sem):
            lists = []
            for g in range(n_groups):
                blk, sub = g // subs, g % subs
                v = mv[blk * rows_per_w + i, pl.ds(sub * 16, 16)]
                ids = iota + (g * 16)
                lists.append(_sort16(v, ids))
            l32 = []
            for p in range(n_groups // 2):
                a, b = lists[2 * p], lists[2 * p + 1]
                l32.append(_sort2_full(a[0], a[1], b[0], b[1]))
            while len(l32) > 1:
                nxt = []
                for p in range(len(l32) // 2):
                    nxt.append(_merge32(*l32[2 * p], *l32[2 * p + 1]))
                l32 = nxt
            tv0, ti0, tv1, ti1 = l32[0]

            t_splat = jnp.take(tv1, jnp.full((16,), k - 17, jnp.int32))
            cnt = (16 + plsc.all_reduce_population_count(tv1 >= t_splat)[0])

            row_off = (base + i) * n_chunks
            pltpu.async_copy(sim_hbm.at[row_off + ti0],
                             gbuf.at[pl.ds(0, 16)], sem)
            pltpu.async_copy(sim_hbm.at[row_off + ti1],
                             gbuf.at[pl.ds(16, 16)], sem)
            return ti0, ti1, t_splat, cnt

        def process(i, gbuf, sem, st):
            ti0, ti1, t_splat, cnt = st
            m_splat = msv[i, pl.ds(0, 16)]
            s_splat = ssv[i, pl.ds(0, 16)]

            pltpu.make_async_copy(sim_hbm.at[pl.ds(0, 32)], gbuf, sem).wait()

            def chunk_body(q, carry):
                rv0, ri0, rv1, ri1 = carry
                use0 = jnp.full((16,), q < 16, jnp.bool_)
                ti = jnp.where(use0, ti0, ti1)
                cid = jnp.take(ti, jnp.full((16,), q % 16, jnp.int32))
                colbase = cid * CHUNK
                for jj in range(CHUNK // 16):
                    v = gbuf[q, pl.ds(jj * 16, 16)]
                    gidx = colbase + (jj * 16) + iota
                    mask = v >= t_splat
                    pc = plsc.all_reduce_population_count(mask)[0]

                    def do_merge(rv0=rv0, ri0=ri0, rv1=rv1, ri1=ri1,
                                 v=v, gidx=gidx, mask=mask):
                        sv, si = _sort16(jnp.where(mask, v, NEG_INF), gidx)
                        return _merge32(rv0, ri0, rv1, ri1,
                                        sv, si, neg_inf_v, zero_i)

                    def no_merge(rv0=rv0, ri0=ri0, rv1=rv1, ri1=ri1):
                        return rv0, ri0, rv1, ri1

                    rv0, ri0, rv1, ri1 = lax.cond(pc > 0, do_merge, no_merge)
                return rv0, ri0, rv1, ri1

            rv0, ri0, rv1, ri1 = lax.fori_loop(
                0, cnt, chunk_body, (neg_inf_v, zero_i, neg_inf_v, zero_i))

            sh0 = jnp.where(iota == 15,
                            jnp.take(rv1, jnp.full((16,), 0, jnp.int32)),
                            jnp.take(rv0, jnp.minimum(iota + 1, 15)))
            sh1 = jnp.take(rv1, jnp.minimum(iota + 1, 15))
            t0 = plsc.all_reduce_population_count(rv0 == sh0)[0]
            t1 = plsc.all_reduce_population_count(
                (rv1 == sh1) & (iota < 5))[0]

            def fast(rv0=rv0, ri0=ri0, rv1=rv1, ri1=ri1):
                sc0 = jnp.exp(rv0 - m_splat) / s_splat
                sc1 = jnp.exp(rv1 - m_splat) / s_splat
                return sc0, ri0, sc1, ri1

            def slow(rv0=rv0, ri0=ri0, rv1=rv1, ri1=ri1):
                big = jnp.int32(2 ** 30)
                sc_a = neg_inf_v
                sc_b = neg_inf_v
                ix_a = zero_i
                ix_b = zero_i
                for rnd in range(k):
                    gm = jnp.max(jnp.maximum(rv0, rv1))
                    gm_splat = jnp.full((16,), gm, jnp.float32)
                    c0 = rv0 == gm_splat
                    c1m = rv1 == gm_splat
                    gidx = jnp.min(jnp.minimum(jnp.where(c0, ri0, big),
                                               jnp.where(c1m, ri1, big)))
                    gidx_splat = jnp.full((16,), gidx, jnp.int32)
                    score = jnp.exp(gm_splat - m_splat) / s_splat
                    lane = iota == (rnd % 16)
                    if rnd < 16:
                        sc_a = jnp.where(lane, score, sc_a)
                        ix_a = jnp.where(lane, gidx_splat, ix_a)
                    else:
                        sc_b = jnp.where(lane, score, sc_b)
                        ix_b = jnp.where(lane, gidx_splat, ix_b)
                    rv0 = jnp.where(c0 & (ri0 == gidx_splat), NEG_INF, rv0)
                    rv1 = jnp.where(c1m & (ri1 == gidx_splat), NEG_INF, rv1)
                return sc_a, ix_a, sc_b, ix_b

            sc_a, ix_a, sc_b, ix_b = lax.cond(t0 + t1 > 0, slow, fast)

            ov[i, pl.ds(0, 16)] = sc_a
            ov[i, pl.ds(16, 16)] = sc_b
            oi[i, pl.ds(0, 16)] = ix_a
            oi[i, pl.ds(16, 16)] = ix_b

        st_a0 = phase_a(jnp.int32(0), gbuf_a, sem_a)
        st_b0 = phase_a(jnp.int32(1), gbuf_b, sem_b)

        def pipe_body(t, carry):
            st_a, st_b = carry
            i = 2 * t
            process(i, gbuf_a, sem_a, st_a)
            st_a = phase_a(i + 2, gbuf_a, sem_a)
            process(i + 1, gbuf_b, sem_b, st_b)
            st_b = phase_a(i + 3, gbuf_b, sem_b)
            return st_a, st_b

        st_a, st_b = lax.fori_loop(0, rows_per_w // 2 - 1, pipe_body,
                                   (st_a0, st_b0))
        process(jnp.int32(rows_per_w - 2), gbuf_a, sem_a, st_a)
        process(jnp.int32(rows_per_w - 1), gbuf_b, sem_b, st_b)

        pltpu.sync_copy(ov, osc_hbm.at[pl.ds(base, rows_per_w)])
        pltpu.sync_copy(oi, oix_hbm.at[pl.ds(base, rows_per_w)])

    return body(sim2d, mmax, m2, s2)


@jax.jit
def kernel(queries, db):
    q_n, _ = queries.shape
    n, _ = db.shape
    k = min(K_TOP, n)
    n_chunks = n // CHUNK

    h = q_n // 2
    tc_outs = [_tc_stage(queries[p * h:(p + 1) * h], db) for p in range(2)]
    sc_outs = []
    for sim3d, m2, s2, m3 in tc_outs:
        sim2d = jnp.reshape(sim3d, (h * n_chunks, CHUNK))
        mmax = jnp.reshape(m3, (m3.shape[0] * h, 128))
        sc_outs.append(_sc_stage(sim2d, mmax, m2, s2, q_n=h,
                                 n_chunks=n_chunks, n_blocks=m3.shape[0],
                                 k=k))
    oscores = jnp.concatenate([o[0] for o in sc_outs], axis=0)
    oinds = jnp.concatenate([o[1] for o in sc_outs], axis=0)

    rows = jnp.arange(q_n * k, dtype=jnp.int32) // k
    return rows, oinds[:, :k].reshape(-1), oscores[:, :k].reshape(-1)
